# Initial kernel scaffold; baseline (speedup 1.0000x reference)
import functools

import jax
import jax.numpy as jnp
from jax import lax
from jax.experimental import pallas as pl
from jax.experimental.pallas import tpu as pltpu
from jax.experimental.pallas import tpu_sc as plsc

B, C, H, W = 32, 80, 128, 128
K = 100
HW = H * W
NROW = C * H
NVR = NROW // 16
NGJ = 32
NGT = NVR // NGJ
ROWCAP = 512
ELCAP = 512
OUTW = 128
CB = 8

_I32_MAX = jnp.int32(2**31 - 1)
_MASK31 = jnp.int32(0x7FFFFFFF)


def _rowmax_body(x_ref, o_ref):
    o_ref[...] = jnp.max(x_ref[...], axis=-1)


def _f32_key(v):
    kb = plsc.bitcast(v, jnp.int32)
    return jnp.where(kb >= 0, kb, kb ^ _MASK31)


def _key_f32(k):
    return plsc.bitcast(jnp.where(k >= 0, k, k ^ _MASK31), jnp.float32)


def _splat(x, dtype=jnp.int32):
    return jnp.full((16,), x, dtype)


def _scalar(vec, is_min=False):
    return jnp.min(vec, axis=0) if is_min else jnp.max(vec, axis=0)


def _select_body(rm_hbm, sc2_hbm, ov_hbm, oi_hbm, oc_hbm,
                 rm, gk, cand, gidx, gbuf, fval, fpos, sval, spos, scls, sem):
    nc = plsc.get_sparse_core_info().num_cores
    b = lax.axis_index("s") * nc + lax.axis_index("c")
    iota = lax.iota(jnp.int32, 16)
    neg_inf = _splat(-jnp.inf, jnp.float32)

    pltpu.sync_copy(rm_hbm.at[b], rm)

    def _gmax(t, accs):
        return tuple(
            jnp.maximum(accs[j], rm[pl.ds((j + NGJ * t) * 16, 16)])
            for j in range(NGJ))

    accs = lax.fori_loop(1, NGT, _gmax,
                         tuple(rm[pl.ds(j * 16, 16)] for j in range(NGJ)))
    for j in range(NGJ):
        gk[pl.ds(j * 16, 16)] = _f32_key(accs[j])

    def _count_ge(t_splat):
        def body(i, acc):
            return acc + plsc.all_reduce_population_count(
                gk[pl.ds(i * 16, 16)] >= t_splat)
        return lax.fori_loop(0, NGJ, body, _splat(0))

    ge0 = _count_ge(_splat(0)) >= K
    lo = jnp.where(ge0, _splat(0), _splat(jnp.int32(-2**31)))
    hi = jnp.where(ge0, _splat(_I32_MAX), _splat(-1))

    def _bis(_, carry):
        lo, hi = carry
        d = hi - lo
        mid = lo + (d >> 1) + (d & 1)
        ge = _count_ge(mid) >= K
        return jnp.where(ge, mid, lo), jnp.where(ge, hi, mid - 1)

    lo, hi = lax.fori_loop(0, 31, _bis, (lo, hi))
    thr = _key_f32(lo)

    def _zero(ref, val, n):
        def body(j, _):
            ref[pl.ds(j * 16, 16)] = val
            return 0
        lax.fori_loop(0, n, body, 0)

    _zero(cand, _splat(0), ROWCAP // 16)

    def _rowscan(i, off):
        m = rm[pl.ds(i * 16, 16)] >= thr

        def hit(o):
            cnt = _scalar(plsc.all_reduce_population_count(m))
            plsc.store_compressed(
                cand.at[pl.ds(jnp.minimum(o, ROWCAP - 16), 16)],
                iota + i * 16, mask=m)
            return o + cnt

        return lax.cond(jnp.any(m), hit, lambda o: o, off)

    nrows = jnp.minimum(lax.fori_loop(0, NVR, _rowscan, 0), ROWCAP)

    base = b * NROW
    for j in range(NGJ):
        gidx[j // 8, pl.ds((j % 8) * 16, 16)] = cand[pl.ds(j * 16, 16)] + base
    copies = [
        pltpu.async_copy(sc2_hbm.at[gidx.at[q]],
                         gbuf.at[pl.ds(q * 128, 128)], sem)
        for q in range(ROWCAP // 128)
    ]
    for cp in copies:
        cp.wait()

    _zero(fval, neg_inf, ELCAP // 16)
    _zero(fpos, _splat(_I32_MAX), ELCAP // 16)

    def _elscan(s, off):
        rowid = plsc.load_gather(cand, [_splat(s)])
        pos0 = rowid * W

        def inner(w, off):
            v = gbuf[s, pl.ds(w * 16, 16)]
            m = v >= thr

            def hit(o):
                cnt = _scalar(plsc.all_reduce_population_count(m))
                oc = jnp.minimum(o, ELCAP - 16)
                plsc.store_compressed(fval.at[pl.ds(oc, 16)], v, mask=m)
                plsc.store_compressed(fpos.at[pl.ds(oc, 16)],
                                      pos0 + w * 16 + iota, mask=m)
                return o + cnt

            return lax.cond(jnp.any(m), hit, lambda o: o, off)

        return lax.fori_loop(0, W // 16, inner, off)

    nel = jnp.minimum(lax.fori_loop(0, nrows, _elscan, 0), ELCAP)
    nv = (nel + 15) >> 4

    lane0 = iota == 0

    def _ext(k, _):
        def fmax(i, mm):
            return jnp.maximum(mm, fval[pl.ds(i * 16, 16)])
        mval = _splat(_scalar(lax.fori_loop(0, nv, fmax, neg_inf)),
                      jnp.float32)

        def pmin(i, pp):
            sel = fval[pl.ds(i * 16, 16)] == mval
            return jnp.minimum(pp, jnp.where(sel, fpos[pl.ds(i * 16, 16)],
                                             _I32_MAX))
        mpos = _splat(_scalar(lax.fori_loop(0, nv, pmin, _splat(_I32_MAX)),
                              is_min=True))

        def kill(i, _):
            v = fval[pl.ds(i * 16, 16)]
            sel = (v == mval) & (fpos[pl.ds(i * 16, 16)] == mpos)
            fval[pl.ds(i * 16, 16)] = jnp.where(sel, neg_inf, v)
            return 0
        lax.fori_loop(0, nv, kill, 0)

        ks = _splat(k)
        plsc.store_scatter(sval, [ks], mval, mask=lane0)
        plsc.store_scatter(spos, [ks], mpos, mask=lane0)
        return 0

    lax.fori_loop(0, K, _ext, 0)

    for j in range(OUTW // 16):
        sl = pl.ds(j * 16, 16)
        if j * 16 >= K:
            sval[sl] = jnp.zeros((16,), jnp.float32)
            spos[sl] = _splat(0)
        p = spos[sl]
        scls[sl] = p >> 14
        spos[sl] = p & jnp.int32(HW - 1)
    pltpu.sync_copy(sval, ov_hbm.at[b])
    pltpu.sync_copy(spos, oi_hbm.at[b])
    pltpu.sync_copy(scls, oc_hbm.at[b])


@jax.jit
def kernel(scores):
    rowmax = pl.pallas_call(
        _rowmax_body,
        grid=(B, C // CB),
        in_specs=[pl.BlockSpec((1, CB, H, W), lambda b, c: (b, c, 0, 0))],
        out_specs=pl.BlockSpec((1, CB, H), lambda b, c: (b, c, 0)),
        out_shape=jax.ShapeDtypeStruct((B, C, H), jnp.float32),
    )(scores)

    select = functools.partial(
        pl.kernel,
        out_type=[
            jax.ShapeDtypeStruct((B, OUTW), jnp.float32),
            jax.ShapeDtypeStruct((B, OUTW), jnp.int32),
            jax.ShapeDtypeStruct((B, OUTW), jnp.int32),
        ],
        mesh=plsc.VectorSubcoreMesh(core_axis_name="c", subcore_axis_name="s"),
        scratch_types=[
            pltpu.VMEM((NROW,), jnp.float32),
            pltpu.VMEM((NGJ * 16,), jnp.int32),
            pltpu.VMEM((ROWCAP,), jnp.int32),
            pltpu.VMEM((ROWCAP // 128, 128), jnp.int32),
            pltpu.VMEM((ROWCAP, W), jnp.float32),
            pltpu.VMEM((ELCAP,), jnp.float32),
            pltpu.VMEM((ELCAP,), jnp.int32),
            pltpu.VMEM((OUTW,), jnp.float32),
            pltpu.VMEM((OUTW,), jnp.int32),
            pltpu.VMEM((OUTW,), jnp.int32),
            pltpu.SemaphoreType.DMA,
        ],
    )(_select_body)

    ov, oi, oc = select(rowmax.reshape(B, NROW), scores.reshape(B * NROW, W))
    return ov[:, :K], oi[:, :K], oc[:, :K]


# speedup vs baseline: 68.7904x; 68.7904x over previous
import functools

import jax
import jax.numpy as jnp
from jax import lax
from jax.experimental import pallas as pl
from jax.experimental.pallas import tpu as pltpu
from jax.experimental.pallas import tpu_sc as plsc

B, C, H, W = 32, 80, 128, 128
K = 100
HW = H * W
NROW = C * H
NVR = NROW // 16
NGJ = 32
NGT = NVR // NGJ
ROWCAP = 512
ELCAP = 512
OUTW = 128
CB = 8
NCORES = 2
NSUB = 16

_I32_MAX = 2**31 - 1
_MASK31 = 0x7FFFFFFF


def _rowmax_body(x_ref, o_ref):
    o_ref[...] = jnp.max(x_ref[...], axis=-1)


def _f32_key(v):
    kb = lax.bitcast_convert_type(v, jnp.int32)
    return jnp.where(kb >= 0, kb, kb ^ _MASK31)


def _key_f32(k):
    return lax.bitcast_convert_type(jnp.where(k >= 0, k, k ^ _MASK31),
                                    jnp.float32)


def _splat(x, dtype=jnp.int32):
    return jnp.full((16,), x, dtype)


def _scalar(vec, is_min=False):
    return jnp.min(vec, axis=0) if is_min else jnp.max(vec, axis=0)


def _select_body(rm_hbm, sc2_hbm, ov_hbm, oi_hbm, oc_hbm,
                 rm, gk, cand, gidx, gbuf, fval, fpos, sval, spos, scls, sem):
    b = lax.axis_index("s") * NCORES + lax.axis_index("c")
    iota = lax.iota(jnp.int32, 16)
    neg_inf = _splat(-jnp.inf, jnp.float32)

    pltpu.sync_copy(rm_hbm.at[b], rm)

    def _gmax(t, accs):
        return tuple(
            jnp.maximum(accs[j], rm[pl.ds((j + NGJ * t) * 16, 16)])
            for j in range(NGJ))

    accs = lax.fori_loop(1, NGT, _gmax,
                         tuple(rm[pl.ds(j * 16, 16)] for j in range(NGJ)))
    for j in range(NGJ):
        gk[pl.ds(j * 16, 16)] = _f32_key(accs[j])

    def _count_ge(t):
        def body(i, acc):
            m = gk[pl.ds(i * 16, 16)] >= _splat(t)
            return acc + m.astype(jnp.int32)
        return jnp.sum(lax.fori_loop(0, NGJ, body, _splat(0)), axis=0)

    ge0 = _count_ge(jnp.int32(0)) >= K
    lo = jnp.where(ge0, jnp.int32(0), jnp.int32(-2**31))
    hi = jnp.where(ge0, jnp.int32(_I32_MAX), jnp.int32(-1))

    def _bis(_, carry):
        lo, hi = carry
        d = hi - lo
        mid = lo + (d >> 1) + (d & 1)
        ge = _count_ge(mid) >= K
        return jnp.where(ge, mid, lo), jnp.where(ge, hi, mid - 1)

    lo, hi = lax.fori_loop(0, 31, _bis, (lo, hi))
    thr = _key_f32(_splat(lo))

    def _zero(ref, val, n):
        def body(j, _):
            ref[pl.ds(j * 16, 16)] = val
            return 0
        lax.fori_loop(0, n, body, 0)

    _zero(cand, _splat(0), ROWCAP // 16)

    def _rowscan(i, off):
        m = rm[pl.ds(i * 16, 16)] >= thr

        def hit(o):
            cnt = jnp.sum(m.astype(jnp.int32), axis=0)
            plsc.store_compressed(
                cand.at[pl.ds(jnp.minimum(o, ROWCAP - 16), 16)],
                iota + i * 16, mask=m)
            return o + cnt

        return lax.cond(jnp.any(m), hit, lambda o: o, off)

    nrows = jnp.minimum(lax.fori_loop(0, NVR, _rowscan, 0), ROWCAP)

    base = b * NROW
    for j in range(NGJ):
        gidx[j // 8, pl.ds((j % 8) * 16, 16)] = cand[pl.ds(j * 16, 16)] + base
    copies = [
        pltpu.async_copy(sc2_hbm.at[gidx.at[q]],
                         gbuf.at[pl.ds(q * 128, 128)], sem)
        for q in range(ROWCAP // 128)
    ]
    for cp in copies:
        cp.wait()

    _zero(fval, neg_inf, ELCAP // 16)
    _zero(fpos, _splat(_I32_MAX), ELCAP // 16)

    def _elscan(s, off):
        rowid = plsc.load_gather(cand, [_splat(s)])
        pos0 = rowid * W

        def inner(w, off):
            v = gbuf[s, pl.ds(w * 16, 16)]
            m = v >= thr

            def hit(o):
                cnt = jnp.sum(m.astype(jnp.int32), axis=0)
                oc = jnp.minimum(o, ELCAP - 16)
                plsc.store_compressed(fval.at[pl.ds(oc, 16)], v, mask=m)
                plsc.store_compressed(fpos.at[pl.ds(oc, 16)],
                                      pos0 + w * 16 + iota, mask=m)
                return o + cnt

            return lax.cond(jnp.any(m), hit, lambda o: o, off)

        return lax.fori_loop(0, W // 16, inner, off)

    nel = jnp.minimum(lax.fori_loop(0, nrows, _elscan, 0), ELCAP)
    nv = (nel + 15) >> 4

    lane0 = iota == 0

    def _ext(k, _):
        def fmax(i, mm):
            return jnp.maximum(mm, fval[pl.ds(i * 16, 16)])
        mval = _splat(_scalar(lax.fori_loop(0, nv, fmax, neg_inf)),
                      jnp.float32)

        def pmin(i, pp):
            sel = fval[pl.ds(i * 16, 16)] == mval
            return jnp.minimum(pp, jnp.where(sel, fpos[pl.ds(i * 16, 16)],
                                             _I32_MAX))
        mpos = _splat(_scalar(lax.fori_loop(0, nv, pmin, _splat(_I32_MAX)),
                              is_min=True))

        def kill(i, _):
            v = fval[pl.ds(i * 16, 16)]
            sel = (v == mval) & (fpos[pl.ds(i * 16, 16)] == mpos)
            fval[pl.ds(i * 16, 16)] = jnp.where(sel, neg_inf, v)
            return 0
        lax.fori_loop(0, nv, kill, 0)

        ks = _splat(k)
        plsc.store_scatter(sval, [ks], mval, mask=lane0)
        plsc.store_scatter(spos, [ks], mpos, mask=lane0)
        return 0

    lax.fori_loop(0, K, _ext, 0)

    for j in range(OUTW // 16):
        sl = pl.ds(j * 16, 16)
        if j * 16 >= K:
            sval[sl] = jnp.zeros((16,), jnp.float32)
            spos[sl] = _splat(0)
        p = spos[sl]
        scls[sl] = p >> 14
        spos[sl] = p & (HW - 1)
    pltpu.sync_copy(sval, ov_hbm.at[b])
    pltpu.sync_copy(spos, oi_hbm.at[b])
    pltpu.sync_copy(scls, oc_hbm.at[b])


@jax.jit
def kernel(scores):
    rowmax = pl.pallas_call(
        _rowmax_body,
        grid=(B, C // CB),
        in_specs=[pl.BlockSpec((1, CB, H, W), lambda b, c: (b, c, 0, 0))],
        out_specs=pl.BlockSpec((1, CB, H), lambda b, c: (b, c, 0)),
        out_shape=jax.ShapeDtypeStruct((B, C, H), jnp.float32),
    )(scores)

    select = functools.partial(
        pl.kernel,
        out_type=[
            jax.ShapeDtypeStruct((B, OUTW), jnp.float32),
            jax.ShapeDtypeStruct((B, OUTW), jnp.int32),
            jax.ShapeDtypeStruct((B, OUTW), jnp.int32),
        ],
        mesh=plsc.VectorSubcoreMesh(core_axis_name="c", subcore_axis_name="s",
                                    num_cores=NCORES, num_subcores=NSUB),
        compiler_params=pltpu.CompilerParams(needs_layout_passes=False),
        scratch_types=[
            pltpu.VMEM((NROW,), jnp.float32),
            pltpu.VMEM((NGJ * 16,), jnp.int32),
            pltpu.VMEM((ROWCAP,), jnp.int32),
            pltpu.VMEM((ROWCAP // 128, 128), jnp.int32),
            pltpu.VMEM((ROWCAP, W), jnp.float32),
            pltpu.VMEM((ELCAP,), jnp.float32),
            pltpu.VMEM((ELCAP,), jnp.int32),
            pltpu.VMEM((OUTW,), jnp.float32),
            pltpu.VMEM((OUTW,), jnp.int32),
            pltpu.VMEM((OUTW,), jnp.int32),
            pltpu.SemaphoreType.DMA,
        ],
    )(_select_body)

    ov, oi, oc = select(rowmax.reshape(B, NROW), scores.reshape(B * NROW, W))
    return ov[:, :K], oi[:, :K], oc[:, :K]


# speedup vs baseline: 91.4507x; 1.3294x over previous
import functools

import jax
import jax.numpy as jnp
from jax import lax
from jax.experimental import pallas as pl
from jax.experimental.pallas import tpu as pltpu
from jax.experimental.pallas import tpu_sc as plsc

B, C, H, W = 32, 80, 128, 128
K = 100
HW = H * W
NROW = C * H
NVR = NROW // 16
NGJ = 32
NGT = NVR // NGJ
ROWCAP = 512
ELCAP = 512
OUTW = 128
CB = 16
NCORES = 2
NSUB = 16

_I32_MAX = 2**31 - 1
_MASK31 = 0x7FFFFFFF


def _rowmax_body(x_ref, o_ref):
    o_ref[...] = jnp.max(x_ref[...], axis=-1)


def _f32_key(v):
    kb = lax.bitcast_convert_type(v, jnp.int32)
    return jnp.where(kb >= 0, kb, kb ^ _MASK31)


def _key_f32(k):
    return lax.bitcast_convert_type(jnp.where(k >= 0, k, k ^ _MASK31),
                                    jnp.float32)


def _splat(x, dtype=jnp.int32):
    return jnp.full((16,), x, dtype)


def _scalar(vec, is_min=False):
    return jnp.min(vec, axis=0) if is_min else jnp.max(vec, axis=0)


def _select_body(rm_hbm, sc2_hbm, ov_hbm, oi_hbm, oc_hbm,
                 rm, gk, cand, gidx, gbuf, fval, fpos, sval, spos, scls, sem):
    b = lax.axis_index("s") * NCORES + lax.axis_index("c")
    iota = lax.iota(jnp.int32, 16)
    neg_inf = _splat(-jnp.inf, jnp.float32)

    pltpu.sync_copy(rm_hbm.at[b], rm)

    def _gmax(t, accs):
        return tuple(
            jnp.maximum(accs[j], rm[pl.ds((j + NGJ * t) * 16, 16)])
            for j in range(NGJ))

    accs = lax.fori_loop(1, NGT, _gmax,
                         tuple(rm[pl.ds(j * 16, 16)] for j in range(NGJ)))
    for j in range(NGJ):
        gk[pl.ds(j * 16, 16)] = _f32_key(accs[j])

    def _count_ge(t):
        def body(i, acc):
            m = gk[pl.ds(i * 16, 16)] >= _splat(t)
            return acc + m.astype(jnp.int32)
        return jnp.sum(lax.fori_loop(0, NGJ, body, _splat(0)), axis=0)

    ge0 = _count_ge(jnp.int32(0)) >= K
    lo = jnp.where(ge0, jnp.int32(0), jnp.int32(-2**31))
    hi = jnp.where(ge0, jnp.int32(_I32_MAX), jnp.int32(-1))

    def _bis(_, carry):
        lo, hi = carry
        d = hi - lo
        mid = lo + (d >> 1) + (d & 1)
        ge = _count_ge(mid) >= K
        return jnp.where(ge, mid, lo), jnp.where(ge, hi, mid - 1)

    lo, hi = lax.fori_loop(0, 31, _bis, (lo, hi))
    thr = _key_f32(_splat(lo))

    def _zero(ref, val, n):
        def body(j, _):
            ref[pl.ds(j * 16, 16)] = val
            return 0
        lax.fori_loop(0, n, body, 0)

    _zero(cand, _splat(0), ROWCAP // 16)

    def _rowscan(i, off):
        m = rm[pl.ds(i * 16, 16)] >= thr

        def hit(o):
            cnt = jnp.sum(m.astype(jnp.int32), axis=0)
            plsc.store_compressed(
                cand.at[pl.ds(jnp.minimum(o, ROWCAP - 16), 16)],
                iota + i * 16, mask=m)
            return o + cnt

        return lax.cond(jnp.any(m), hit, lambda o: o, off)

    nrows = jnp.minimum(lax.fori_loop(0, NVR, _rowscan, 0), ROWCAP)

    base = b * NROW
    for j in range(NGJ):
        gidx[j // 8, pl.ds((j % 8) * 16, 16)] = cand[pl.ds(j * 16, 16)] + base
    copies = [
        pltpu.async_copy(sc2_hbm.at[gidx.at[q]],
                         gbuf.at[pl.ds(q * 128, 128)], sem)
        for q in range(ROWCAP // 128)
    ]
    for cp in copies:
        cp.wait()

    _zero(fval, neg_inf, ELCAP // 16)
    _zero(fpos, _splat(_I32_MAX), ELCAP // 16)

    def _elscan(s, off):
        rowid = plsc.load_gather(cand, [_splat(s)])
        pos0 = rowid * W

        def inner(w, off):
            v = gbuf[s, pl.ds(w * 16, 16)]
            m = v >= thr

            def hit(o):
                cnt = jnp.sum(m.astype(jnp.int32), axis=0)
                oc = jnp.minimum(o, ELCAP - 16)
                plsc.store_compressed(fval.at[pl.ds(oc, 16)], v, mask=m)
                plsc.store_compressed(fpos.at[pl.ds(oc, 16)],
                                      pos0 + w * 16 + iota, mask=m)
                return o + cnt

            return lax.cond(jnp.any(m), hit, lambda o: o, off)

        return lax.fori_loop(0, W // 16, inner, off)

    nel = jnp.minimum(lax.fori_loop(0, nrows, _elscan, 0), ELCAP)
    nv = (nel + 15) >> 4

    lane0 = iota == 0

    def _ext(k, _):
        def fmax(i, mm):
            return jnp.maximum(mm, fval[pl.ds(i * 16, 16)])
        mval = _splat(_scalar(lax.fori_loop(0, nv, fmax, neg_inf)),
                      jnp.float32)

        def pmin(i, pp):
            sel = fval[pl.ds(i * 16, 16)] == mval
            return jnp.minimum(pp, jnp.where(sel, fpos[pl.ds(i * 16, 16)],
                                             _I32_MAX))
        mpos = _splat(_scalar(lax.fori_loop(0, nv, pmin, _splat(_I32_MAX)),
                              is_min=True))

        def kill(i, _):
            v = fval[pl.ds(i * 16, 16)]
            sel = (v == mval) & (fpos[pl.ds(i * 16, 16)] == mpos)
            fval[pl.ds(i * 16, 16)] = jnp.where(sel, neg_inf, v)
            return 0
        lax.fori_loop(0, nv, kill, 0)

        ks = _splat(k)
        plsc.store_scatter(sval, [ks], mval, mask=lane0)
        plsc.store_scatter(spos, [ks], mpos, mask=lane0)
        return 0

    lax.fori_loop(0, K, _ext, 0)

    for j in range(OUTW // 16):
        sl = pl.ds(j * 16, 16)
        if j * 16 >= K:
            sval[sl] = jnp.zeros((16,), jnp.float32)
            spos[sl] = _splat(0)
        p = spos[sl]
        scls[sl] = p >> 14
        spos[sl] = p & (HW - 1)
    pltpu.sync_copy(sval, ov_hbm.at[b])
    pltpu.sync_copy(spos, oi_hbm.at[b])
    pltpu.sync_copy(scls, oc_hbm.at[b])


@jax.jit
def kernel(scores):
    rowmax = pl.pallas_call(
        _rowmax_body,
        grid=(B, C // CB),
        in_specs=[pl.BlockSpec((1, CB, H, W), lambda b, c: (b, c, 0, 0))],
        out_specs=pl.BlockSpec((1, CB, H), lambda b, c: (b, c, 0)),
        out_shape=jax.ShapeDtypeStruct((B, C, H), jnp.float32),
    )(scores)

    select = functools.partial(
        pl.kernel,
        out_type=[
            jax.ShapeDtypeStruct((B, OUTW), jnp.float32),
            jax.ShapeDtypeStruct((B, OUTW), jnp.int32),
            jax.ShapeDtypeStruct((B, OUTW), jnp.int32),
        ],
        mesh=plsc.VectorSubcoreMesh(core_axis_name="c", subcore_axis_name="s",
                                    num_cores=NCORES, num_subcores=NSUB),
        compiler_params=pltpu.CompilerParams(needs_layout_passes=False),
        scratch_types=[
            pltpu.VMEM((NROW,), jnp.float32),
            pltpu.VMEM((NGJ * 16,), jnp.int32),
            pltpu.VMEM((ROWCAP,), jnp.int32),
            pltpu.VMEM((ROWCAP // 128, 128), jnp.int32),
            pltpu.VMEM((ROWCAP, W), jnp.float32),
            pltpu.VMEM((ELCAP,), jnp.float32),
            pltpu.VMEM((ELCAP,), jnp.int32),
            pltpu.VMEM((OUTW,), jnp.float32),
            pltpu.VMEM((OUTW,), jnp.int32),
            pltpu.VMEM((OUTW,), jnp.int32),
            pltpu.SemaphoreType.DMA,
        ],
    )(_select_body)

    ov, oi, oc = select(rowmax.reshape(B, NROW), scores.reshape(B * NROW, W))
    return ov[:, :K], oi[:, :K], oc[:, :K]


# speedup vs baseline: 113.2599x; 1.2385x over previous
import functools

import jax
import jax.numpy as jnp
from jax import lax
from jax.experimental import pallas as pl
from jax.experimental.pallas import tpu as pltpu
from jax.experimental.pallas import tpu_sc as plsc

B, C, H, W = 32, 80, 128, 128
K = 100
HW = H * W
NROW = C * H
NVR = NROW // 16
NGJ = 32
NGT = NVR // NGJ
ROWCAP = 512
ELCAP = 512
OUTW = 128
CB = 40
NCORES = 2
NSUB = 16

_I32_MAX = 2**31 - 1
_MASK31 = 0x7FFFFFFF


def _rowmax_body(x_ref, o_ref):
    o_ref[...] = jnp.max(x_ref[...], axis=-1)


def _f32_key(v):
    kb = lax.bitcast_convert_type(v, jnp.int32)
    return jnp.where(kb >= 0, kb, kb ^ _MASK31)


def _key_f32(k):
    return lax.bitcast_convert_type(jnp.where(k >= 0, k, k ^ _MASK31),
                                    jnp.float32)


def _splat(x, dtype=jnp.int32):
    return jnp.full((16,), x, dtype)


def _scalar(vec, is_min=False):
    return jnp.min(vec, axis=0) if is_min else jnp.max(vec, axis=0)


def _select_body(rm_hbm, sc2_hbm, ov_hbm, oi_hbm, oc_hbm,
                 rm, gk, cand, gidx, gbuf, fval, fpos, sval, spos, scls, sem):
    b = lax.axis_index("s") * NCORES + lax.axis_index("c")
    iota = lax.iota(jnp.int32, 16)
    neg_inf = _splat(-jnp.inf, jnp.float32)

    pltpu.sync_copy(rm_hbm.at[b], rm)

    def _gmax(t, accs):
        return tuple(
            jnp.maximum(accs[j], rm[pl.ds((j + NGJ * t) * 16, 16)])
            for j in range(NGJ))

    accs = lax.fori_loop(1, NGT, _gmax,
                         tuple(rm[pl.ds(j * 16, 16)] for j in range(NGJ)))
    for j in range(NGJ):
        gk[pl.ds(j * 16, 16)] = _f32_key(accs[j])

    def _count_ge(t):
        def body(i, acc):
            m = gk[pl.ds(i * 16, 16)] >= _splat(t)
            return acc + m.astype(jnp.int32)
        return jnp.sum(lax.fori_loop(0, NGJ, body, _splat(0)), axis=0)

    ge0 = _count_ge(jnp.int32(0)) >= K
    lo = jnp.where(ge0, jnp.int32(0), jnp.int32(-2**31))
    hi = jnp.where(ge0, jnp.int32(_I32_MAX), jnp.int32(-1))

    def _bis(_, carry):
        lo, hi = carry
        d = hi - lo
        mid = lo + (d >> 1) + (d & 1)
        ge = _count_ge(mid) >= K
        return jnp.where(ge, mid, lo), jnp.where(ge, hi, mid - 1)

    lo, hi = lax.fori_loop(0, 31, _bis, (lo, hi))
    thr = _key_f32(_splat(lo))

    def _zero(ref, val, n):
        def body(j, _):
            ref[pl.ds(j * 16, 16)] = val
            return 0
        lax.fori_loop(0, n, body, 0)

    _zero(cand, _splat(0), ROWCAP // 16)

    def _rowscan(i, off):
        m = rm[pl.ds(i * 16, 16)] >= thr

        def hit(o):
            cnt = jnp.sum(m.astype(jnp.int32), axis=0)
            plsc.store_compressed(
                cand.at[pl.ds(jnp.minimum(o, ROWCAP - 16), 16)],
                iota + i * 16, mask=m)
            return o + cnt

        return lax.cond(jnp.any(m), hit, lambda o: o, off)

    nrows = jnp.minimum(lax.fori_loop(0, NVR, _rowscan, 0), ROWCAP)

    base = b * NROW
    for j in range(NGJ):
        gidx[j // 8, pl.ds((j % 8) * 16, 16)] = cand[pl.ds(j * 16, 16)] + base
    copies = [
        pltpu.async_copy(sc2_hbm.at[gidx.at[q]],
                         gbuf.at[pl.ds(q * 128, 128)], sem)
        for q in range(ROWCAP // 128)
    ]
    for cp in copies:
        cp.wait()

    _zero(fval, neg_inf, ELCAP // 16)
    _zero(fpos, _splat(_I32_MAX), ELCAP // 16)

    def _elscan(s, off):
        rowid = plsc.load_gather(cand, [_splat(s)])
        pos0 = rowid * W

        def inner(w, off):
            v = gbuf[s, pl.ds(w * 16, 16)]
            m = v >= thr

            def hit(o):
                cnt = jnp.sum(m.astype(jnp.int32), axis=0)
                oc = jnp.minimum(o, ELCAP - 16)
                plsc.store_compressed(fval.at[pl.ds(oc, 16)], v, mask=m)
                plsc.store_compressed(fpos.at[pl.ds(oc, 16)],
                                      pos0 + w * 16 + iota, mask=m)
                return o + cnt

            return lax.cond(jnp.any(m), hit, lambda o: o, off)

        return lax.fori_loop(0, W // 16, inner, off)

    nel = jnp.minimum(lax.fori_loop(0, nrows, _elscan, 0), ELCAP)
    nv = (nel + 15) >> 4

    lane0 = iota == 0

    def _ext(k, _):
        def fmax(i, mm):
            return jnp.maximum(mm, fval[pl.ds(i * 16, 16)])
        mval = _splat(_scalar(lax.fori_loop(0, nv, fmax, neg_inf)),
                      jnp.float32)

        def pmin(i, pp):
            sel = fval[pl.ds(i * 16, 16)] == mval
            return jnp.minimum(pp, jnp.where(sel, fpos[pl.ds(i * 16, 16)],
                                             _I32_MAX))
        mpos = _splat(_scalar(lax.fori_loop(0, nv, pmin, _splat(_I32_MAX)),
                              is_min=True))

        def kill(i, _):
            v = fval[pl.ds(i * 16, 16)]
            sel = (v == mval) & (fpos[pl.ds(i * 16, 16)] == mpos)
            fval[pl.ds(i * 16, 16)] = jnp.where(sel, neg_inf, v)
            return 0
        lax.fori_loop(0, nv, kill, 0)

        ks = _splat(k)
        plsc.store_scatter(sval, [ks], mval, mask=lane0)
        plsc.store_scatter(spos, [ks], mpos, mask=lane0)
        return 0

    lax.fori_loop(0, K, _ext, 0)

    for j in range(OUTW // 16):
        sl = pl.ds(j * 16, 16)
        if j * 16 >= K:
            sval[sl] = jnp.zeros((16,), jnp.float32)
            spos[sl] = _splat(0)
        p = spos[sl]
        scls[sl] = p >> 14
        spos[sl] = p & (HW - 1)
    pltpu.sync_copy(sval, ov_hbm.at[b])
    pltpu.sync_copy(spos, oi_hbm.at[b])
    pltpu.sync_copy(scls, oc_hbm.at[b])


@jax.jit
def kernel(scores):
    rowmax = pl.pallas_call(
        _rowmax_body,
        grid=(B, C // CB),
        in_specs=[pl.BlockSpec((1, CB, H, W), lambda b, c: (b, c, 0, 0))],
        out_specs=pl.BlockSpec((1, CB, H), lambda b, c: (b, c, 0)),
        out_shape=jax.ShapeDtypeStruct((B, C, H), jnp.float32),
    )(scores)

    select = functools.partial(
        pl.kernel,
        out_type=[
            jax.ShapeDtypeStruct((B, OUTW), jnp.float32),
            jax.ShapeDtypeStruct((B, OUTW), jnp.int32),
            jax.ShapeDtypeStruct((B, OUTW), jnp.int32),
        ],
        mesh=plsc.VectorSubcoreMesh(core_axis_name="c", subcore_axis_name="s",
                                    num_cores=NCORES, num_subcores=NSUB),
        compiler_params=pltpu.CompilerParams(needs_layout_passes=False),
        scratch_types=[
            pltpu.VMEM((NROW,), jnp.float32),
            pltpu.VMEM((NGJ * 16,), jnp.int32),
            pltpu.VMEM((ROWCAP,), jnp.int32),
            pltpu.VMEM((ROWCAP // 128, 128), jnp.int32),
            pltpu.VMEM((ROWCAP, W), jnp.float32),
            pltpu.VMEM((ELCAP,), jnp.float32),
            pltpu.VMEM((ELCAP,), jnp.int32),
            pltpu.VMEM((OUTW,), jnp.float32),
            pltpu.VMEM((OUTW,), jnp.int32),
            pltpu.VMEM((OUTW,), jnp.int32),
            pltpu.SemaphoreType.DMA,
        ],
    )(_select_body)

    ov, oi, oc = select(rowmax.reshape(B, NROW), scores.reshape(B * NROW, W))
    return ov[:, :K], oi[:, :K], oc[:, :K]


# speedup vs baseline: 125.2146x; 1.1056x over previous
import functools

import jax
import jax.numpy as jnp
from jax import lax
from jax.experimental import pallas as pl
from jax.experimental.pallas import tpu as pltpu
from jax.experimental.pallas import tpu_sc as plsc

B, C, H, W = 32, 80, 128, 128
K = 100
HW = H * W
NROW = C * H
NVR = NROW // 16
NGJ = 32
NGT = NVR // NGJ
ROWCAP = 512
ELCAP = 512
OUTW = 128
CB = 80
NCORES = 2
NSUB = 16

_I32_MAX = 2**31 - 1
_MASK31 = 0x7FFFFFFF


def _rowmax_body(x_ref, o_ref):
    o_ref[...] = jnp.max(x_ref[...], axis=-1)


def _f32_key(v):
    kb = lax.bitcast_convert_type(v, jnp.int32)
    return jnp.where(kb >= 0, kb, kb ^ _MASK31)


def _key_f32(k):
    return lax.bitcast_convert_type(jnp.where(k >= 0, k, k ^ _MASK31),
                                    jnp.float32)


def _splat(x, dtype=jnp.int32):
    return jnp.full((16,), x, dtype)


def _scalar(vec, is_min=False):
    return jnp.min(vec, axis=0) if is_min else jnp.max(vec, axis=0)


def _select_body(rm_hbm, sc2_hbm, ov_hbm, oi_hbm, oc_hbm,
                 rm, gk, cand, gidx, gbuf, fval, fpos, sval, spos, scls, sem):
    b = lax.axis_index("s") * NCORES + lax.axis_index("c")
    iota = lax.iota(jnp.int32, 16)
    neg_inf = _splat(-jnp.inf, jnp.float32)

    pltpu.sync_copy(rm_hbm.at[b], rm)

    def _gmax(t, accs):
        return tuple(
            jnp.maximum(accs[j], rm[pl.ds((j + NGJ * t) * 16, 16)])
            for j in range(NGJ))

    accs = lax.fori_loop(1, NGT, _gmax,
                         tuple(rm[pl.ds(j * 16, 16)] for j in range(NGJ)))
    for j in range(NGJ):
        gk[pl.ds(j * 16, 16)] = _f32_key(accs[j])

    def _count_ge(t):
        def body(i, acc):
            m = gk[pl.ds(i * 16, 16)] >= _splat(t)
            return acc + m.astype(jnp.int32)
        return jnp.sum(lax.fori_loop(0, NGJ, body, _splat(0)), axis=0)

    ge0 = _count_ge(jnp.int32(0)) >= K
    lo = jnp.where(ge0, jnp.int32(0), jnp.int32(-2**31))
    hi = jnp.where(ge0, jnp.int32(_I32_MAX), jnp.int32(-1))

    def _bis(_, carry):
        lo, hi = carry
        d = hi - lo
        mid = lo + (d >> 1) + (d & 1)
        ge = _count_ge(mid) >= K
        return jnp.where(ge, mid, lo), jnp.where(ge, hi, mid - 1)

    lo, hi = lax.fori_loop(0, 31, _bis, (lo, hi))
    thr = _key_f32(_splat(lo))

    def _zero(ref, val, n):
        def body(j, _):
            ref[pl.ds(j * 16, 16)] = val
            return 0
        lax.fori_loop(0, n, body, 0)

    _zero(cand, _splat(0), ROWCAP // 16)

    def _rowscan(i, off):
        m = rm[pl.ds(i * 16, 16)] >= thr

        def hit(o):
            cnt = jnp.sum(m.astype(jnp.int32), axis=0)
            plsc.store_compressed(
                cand.at[pl.ds(jnp.minimum(o, ROWCAP - 16), 16)],
                iota + i * 16, mask=m)
            return o + cnt

        return lax.cond(jnp.any(m), hit, lambda o: o, off)

    nrows = jnp.minimum(lax.fori_loop(0, NVR, _rowscan, 0), ROWCAP)

    base = b * NROW
    for j in range(NGJ):
        gidx[j // 8, pl.ds((j % 8) * 16, 16)] = cand[pl.ds(j * 16, 16)] + base
    copies = [
        pltpu.async_copy(sc2_hbm.at[gidx.at[q]],
                         gbuf.at[pl.ds(q * 128, 128)], sem)
        for q in range(ROWCAP // 128)
    ]
    for cp in copies:
        cp.wait()

    _zero(fval, neg_inf, ELCAP // 16)
    _zero(fpos, _splat(_I32_MAX), ELCAP // 16)

    def _elscan(s, off):
        rowid = plsc.load_gather(cand, [_splat(s)])
        pos0 = rowid * W

        def inner(w, off):
            v = gbuf[s, pl.ds(w * 16, 16)]
            m = v >= thr

            def hit(o):
                cnt = jnp.sum(m.astype(jnp.int32), axis=0)
                oc = jnp.minimum(o, ELCAP - 16)
                plsc.store_compressed(fval.at[pl.ds(oc, 16)], v, mask=m)
                plsc.store_compressed(fpos.at[pl.ds(oc, 16)],
                                      pos0 + w * 16 + iota, mask=m)
                return o + cnt

            return lax.cond(jnp.any(m), hit, lambda o: o, off)

        return lax.fori_loop(0, W // 16, inner, off)

    nel = jnp.minimum(lax.fori_loop(0, nrows, _elscan, 0), ELCAP)
    nv = (nel + 15) >> 4

    lane0 = iota == 0

    def _ext(k, _):
        def fmax(i, mm):
            return jnp.maximum(mm, fval[pl.ds(i * 16, 16)])
        mval = _splat(_scalar(lax.fori_loop(0, nv, fmax, neg_inf)),
                      jnp.float32)

        def pmin(i, pp):
            sel = fval[pl.ds(i * 16, 16)] == mval
            return jnp.minimum(pp, jnp.where(sel, fpos[pl.ds(i * 16, 16)],
                                             _I32_MAX))
        mpos = _splat(_scalar(lax.fori_loop(0, nv, pmin, _splat(_I32_MAX)),
                              is_min=True))

        def kill(i, _):
            v = fval[pl.ds(i * 16, 16)]
            sel = (v == mval) & (fpos[pl.ds(i * 16, 16)] == mpos)
            fval[pl.ds(i * 16, 16)] = jnp.where(sel, neg_inf, v)
            return 0
        lax.fori_loop(0, nv, kill, 0)

        ks = _splat(k)
        plsc.store_scatter(sval, [ks], mval, mask=lane0)
        plsc.store_scatter(spos, [ks], mpos, mask=lane0)
        return 0

    lax.fori_loop(0, K, _ext, 0)

    for j in range(OUTW // 16):
        sl = pl.ds(j * 16, 16)
        if j * 16 >= K:
            sval[sl] = jnp.zeros((16,), jnp.float32)
            spos[sl] = _splat(0)
        p = spos[sl]
        scls[sl] = p >> 14
        spos[sl] = p & (HW - 1)
    pltpu.sync_copy(sval, ov_hbm.at[b])
    pltpu.sync_copy(spos, oi_hbm.at[b])
    pltpu.sync_copy(scls, oc_hbm.at[b])


@jax.jit
def kernel(scores):
    rowmax = pl.pallas_call(
        _rowmax_body,
        grid=(B, C // CB),
        in_specs=[pl.BlockSpec((1, CB, H, W), lambda b, c: (b, c, 0, 0))],
        out_specs=pl.BlockSpec((1, CB, H), lambda b, c: (b, c, 0)),
        out_shape=jax.ShapeDtypeStruct((B, C, H), jnp.float32),
    )(scores)

    select = functools.partial(
        pl.kernel,
        out_type=[
            jax.ShapeDtypeStruct((B, OUTW), jnp.float32),
            jax.ShapeDtypeStruct((B, OUTW), jnp.int32),
            jax.ShapeDtypeStruct((B, OUTW), jnp.int32),
        ],
        mesh=plsc.VectorSubcoreMesh(core_axis_name="c", subcore_axis_name="s",
                                    num_cores=NCORES, num_subcores=NSUB),
        compiler_params=pltpu.CompilerParams(needs_layout_passes=False),
        scratch_types=[
            pltpu.VMEM((NROW,), jnp.float32),
            pltpu.VMEM((NGJ * 16,), jnp.int32),
            pltpu.VMEM((ROWCAP,), jnp.int32),
            pltpu.VMEM((ROWCAP // 128, 128), jnp.int32),
            pltpu.VMEM((ROWCAP, W), jnp.float32),
            pltpu.VMEM((ELCAP,), jnp.float32),
            pltpu.VMEM((ELCAP,), jnp.int32),
            pltpu.VMEM((OUTW,), jnp.float32),
            pltpu.VMEM((OUTW,), jnp.int32),
            pltpu.VMEM((OUTW,), jnp.int32),
            pltpu.SemaphoreType.DMA,
        ],
    )(_select_body)

    ov, oi, oc = select(rowmax.reshape(B, NROW), scores.reshape(B * NROW, W))
    return ov[:, :K], oi[:, :K], oc[:, :K]


# speedup vs baseline: 154.4701x; 1.2336x over previous
import functools

import jax
import jax.numpy as jnp
from jax import lax
from jax.experimental import pallas as pl
from jax.experimental.pallas import tpu as pltpu
from jax.experimental.pallas import tpu_sc as plsc

B, C, H, W = 32, 80, 128, 128
K = 100
HW = H * W
NROW = C * H
NVR = NROW // 16
NGJ = 32
NGT = NVR // NGJ
ROWCAP = 256
ELCAP = 192
NEV = ELCAP // 16
OUTW = 128
CB = 80
NCORES = 2
NSUB = 16

_I32_MAX = 2**31 - 1
_MASK31 = 0x7FFFFFFF


def _rowmax_body(x_ref, o_ref):
    o_ref[...] = jnp.max(x_ref[...], axis=-1)


def _f32_key(v):
    kb = lax.bitcast_convert_type(v, jnp.int32)
    return jnp.where(kb >= 0, kb, kb ^ _MASK31)


def _key_f32(k):
    return lax.bitcast_convert_type(jnp.where(k >= 0, k, k ^ _MASK31),
                                    jnp.float32)


def _splat(x, dtype=jnp.int32):
    return jnp.full((16,), x, dtype)


def _scalar(vec, is_min=False):
    return jnp.min(vec, axis=0) if is_min else jnp.max(vec, axis=0)


def _select_body(rm_hbm, sc2_hbm, ov_hbm, oi_hbm, oc_hbm,
                 rm, gk, cand, gidx, gbuf, fval, fpos, sval, spos, scls, sem):
    b = lax.axis_index("s") * NCORES + lax.axis_index("c")
    iota = lax.iota(jnp.int32, 16)
    neg_inf = _splat(-jnp.inf, jnp.float32)

    pltpu.sync_copy(rm_hbm.at[b], rm)

    def _gmax(t, accs):
        return tuple(
            jnp.maximum(accs[j], rm[pl.ds((j + NGJ * t) * 16, 16)])
            for j in range(NGJ))

    accs = lax.fori_loop(1, NGT, _gmax,
                         tuple(rm[pl.ds(j * 16, 16)] for j in range(NGJ)))
    for j in range(NGJ):
        gk[pl.ds(j * 16, 16)] = _f32_key(accs[j])

    def _count_ge(t):
        ts = _splat(t)
        acc = (gk[pl.ds(0, 16)] >= ts).astype(jnp.int32)
        for i in range(1, NGJ):
            acc = acc + (gk[pl.ds(i * 16, 16)] >= ts).astype(jnp.int32)
        return jnp.sum(acc, axis=0)

    ge0 = _count_ge(jnp.int32(0)) >= K
    lo = jnp.where(ge0, jnp.int32(0), jnp.int32(-2**31))
    hi = jnp.where(ge0, jnp.int32(_I32_MAX), jnp.int32(-1))

    def _bis(_, carry):
        lo, hi = carry
        d = hi - lo
        mid = lo + (d >> 1) + (d & 1)
        ge = _count_ge(mid) >= K
        return jnp.where(ge, mid, lo), jnp.where(ge, hi, mid - 1)

    lo, hi = lax.fori_loop(0, 31, _bis, (lo, hi))
    thr = _key_f32(_splat(lo))

    def _zero(ref, val, n):
        for j in range(n):
            ref[pl.ds(j * 16, 16)] = val

    _zero(cand, _splat(0), ROWCAP // 16)

    RU = 4

    def _rowscan(g, off):
        ms = [rm[pl.ds((g * RU + u) * 16, 16)] >= thr for u in range(RU)]
        anym = ms[0] | ms[1] | ms[2] | ms[3]

        def hit(o):
            for u in range(RU):
                m = ms[u]

                def sub(o, m=m, u=u):
                    cnt = jnp.sum(m.astype(jnp.int32), axis=0)
                    plsc.store_compressed(
                        cand.at[pl.ds(jnp.minimum(o, ROWCAP - 16), 16)],
                        iota + (g * RU + u) * 16, mask=m)
                    return o + cnt

                o = lax.cond(jnp.any(m), sub, lambda o: o, o)
            return o

        return lax.cond(jnp.any(anym), hit, lambda o: o, off)

    nrows = jnp.minimum(lax.fori_loop(0, NVR // RU, _rowscan, 0), ROWCAP)

    base = b * NROW
    for j in range(ROWCAP // 16):
        gidx[j // 8, pl.ds((j % 8) * 16, 16)] = cand[pl.ds(j * 16, 16)] + base
    copies = [
        pltpu.async_copy(sc2_hbm.at[gidx.at[q]],
                         gbuf.at[pl.ds(q * 128, 128)], sem)
        for q in range(ROWCAP // 128)
    ]
    for cp in copies:
        cp.wait()

    _zero(fval, neg_inf, ELCAP // 16)
    _zero(fpos, _splat(_I32_MAX), ELCAP // 16)

    def _elscan(s, off):
        rowid = plsc.load_gather(cand, [_splat(s)])
        pos0 = rowid * W

        def inner(w, off):
            v = gbuf[s, pl.ds(w * 16, 16)]
            m = v >= thr

            def hit(o):
                cnt = jnp.sum(m.astype(jnp.int32), axis=0)
                oc = jnp.minimum(o, ELCAP - 16)
                plsc.store_compressed(fval.at[pl.ds(oc, 16)], v, mask=m)
                plsc.store_compressed(fpos.at[pl.ds(oc, 16)],
                                      pos0 + w * 16 + iota, mask=m)
                return o + cnt

            return lax.cond(jnp.any(m), hit, lambda o: o, off)

        return lax.fori_loop(0, W // 16, inner, off)

    lax.fori_loop(0, nrows, _elscan, 0)

    lane0 = iota == 0

    def _ext(k, _):
        mm = fval[pl.ds(0, 16)]
        for i in range(1, NEV):
            mm = jnp.maximum(mm, fval[pl.ds(i * 16, 16)])
        mval = _splat(_scalar(mm), jnp.float32)

        pp = _splat(_I32_MAX)
        for i in range(NEV):
            sel = fval[pl.ds(i * 16, 16)] == mval
            pp = jnp.minimum(pp, jnp.where(sel, fpos[pl.ds(i * 16, 16)],
                                           _I32_MAX))
        mpos = _splat(_scalar(pp, is_min=True))

        for i in range(NEV):
            v = fval[pl.ds(i * 16, 16)]
            sel = (v == mval) & (fpos[pl.ds(i * 16, 16)] == mpos)
            fval[pl.ds(i * 16, 16)] = jnp.where(sel, neg_inf, v)

        ks = _splat(k)
        plsc.store_scatter(sval, [ks], mval, mask=lane0)
        plsc.store_scatter(spos, [ks], mpos, mask=lane0)
        return 0

    lax.fori_loop(0, K, _ext, 0)

    for j in range(OUTW // 16):
        sl = pl.ds(j * 16, 16)
        if j * 16 >= K:
            sval[sl] = jnp.zeros((16,), jnp.float32)
            spos[sl] = _splat(0)
        p = spos[sl]
        scls[sl] = p >> 14
        spos[sl] = p & (HW - 1)
    pltpu.sync_copy(sval, ov_hbm.at[b])
    pltpu.sync_copy(spos, oi_hbm.at[b])
    pltpu.sync_copy(scls, oc_hbm.at[b])


@jax.jit
def kernel(scores):
    rowmax = pl.pallas_call(
        _rowmax_body,
        grid=(B, C // CB),
        in_specs=[pl.BlockSpec((1, CB, H, W), lambda b, c: (b, c, 0, 0))],
        out_specs=pl.BlockSpec((1, CB, H), lambda b, c: (b, c, 0)),
        out_shape=jax.ShapeDtypeStruct((B, C, H), jnp.float32),
    )(scores)

    select = functools.partial(
        pl.kernel,
        out_type=[
            jax.ShapeDtypeStruct((B, OUTW), jnp.float32),
            jax.ShapeDtypeStruct((B, OUTW), jnp.int32),
            jax.ShapeDtypeStruct((B, OUTW), jnp.int32),
        ],
        mesh=plsc.VectorSubcoreMesh(core_axis_name="c", subcore_axis_name="s",
                                    num_cores=NCORES, num_subcores=NSUB),
        compiler_params=pltpu.CompilerParams(needs_layout_passes=False),
        scratch_types=[
            pltpu.VMEM((NROW,), jnp.float32),
            pltpu.VMEM((NGJ * 16,), jnp.int32),
            pltpu.VMEM((ROWCAP,), jnp.int32),
            pltpu.VMEM((ROWCAP // 128, 128), jnp.int32),
            pltpu.VMEM((ROWCAP, W), jnp.float32),
            pltpu.VMEM((ELCAP,), jnp.float32),
            pltpu.VMEM((ELCAP,), jnp.int32),
            pltpu.VMEM((OUTW,), jnp.float32),
            pltpu.VMEM((OUTW,), jnp.int32),
            pltpu.VMEM((OUTW,), jnp.int32),
            pltpu.SemaphoreType.DMA,
        ],
    )(_select_body)

    ov, oi, oc = select(rowmax.reshape(B, NROW), scores.reshape(B * NROW, W))
    return ov[:, :K], oi[:, :K], oc[:, :K]


# speedup vs baseline: 156.0098x; 1.0100x over previous
import functools

import jax
import jax.numpy as jnp
from jax import lax
from jax.experimental import pallas as pl
from jax.experimental.pallas import tpu as pltpu
from jax.experimental.pallas import tpu_sc as plsc

B, C, H, W = 32, 80, 128, 128
K = 100
HW = H * W
NROW = C * H
NVR = NROW // 16
NGJ = 32
NGT = NVR // NGJ
ROWCAP = 256
ELCAP = 192
NEV = ELCAP // 16
OUTW = 128
CB = 80
NCORES = 2
NSUB = 16

_I32_MAX = 2**31 - 1
_MASK31 = 0x7FFFFFFF


def _rowmax_body(x_ref, o_ref):
    o_ref[...] = jnp.max(x_ref[...], axis=-1)


def _f32_key(v):
    kb = lax.bitcast_convert_type(v, jnp.int32)
    return jnp.where(kb >= 0, kb, kb ^ _MASK31)


def _key_f32(k):
    return lax.bitcast_convert_type(jnp.where(k >= 0, k, k ^ _MASK31),
                                    jnp.float32)


def _splat(x, dtype=jnp.int32):
    return jnp.full((16,), x, dtype)


def _scalar(vec, is_min=False):
    return jnp.min(vec, axis=0) if is_min else jnp.max(vec, axis=0)


def _select_body(rm_hbm, sc2_hbm, ov_hbm, oi_hbm, oc_hbm,
                 rm, gk, cand, gidx, gbuf, fval, fpos, fkey,
                 sval, spos, scls, sem):
    b = lax.axis_index("s") * NCORES + lax.axis_index("c")
    iota = lax.iota(jnp.int32, 16)
    neg_inf = _splat(-jnp.inf, jnp.float32)

    pltpu.sync_copy(rm_hbm.at[b], rm)

    def _gmax(t, accs):
        return tuple(
            jnp.maximum(accs[j], rm[pl.ds((j + NGJ * t) * 16, 16)])
            for j in range(NGJ))

    accs = lax.fori_loop(1, NGT, _gmax,
                         tuple(rm[pl.ds(j * 16, 16)] for j in range(NGJ)))
    for j in range(NGJ):
        gk[pl.ds(j * 16, 16)] = _f32_key(accs[j])

    def _count_ge(t):
        ts = _splat(t)
        acc = (gk[pl.ds(0, 16)] >= ts).astype(jnp.int32)
        for i in range(1, NGJ):
            acc = acc + (gk[pl.ds(i * 16, 16)] >= ts).astype(jnp.int32)
        return jnp.sum(acc, axis=0)

    ge0 = _count_ge(jnp.int32(0)) >= K
    lo = jnp.where(ge0, jnp.int32(0), jnp.int32(-2**31))
    hi = jnp.where(ge0, jnp.int32(_I32_MAX), jnp.int32(-1))

    def _bis(_, carry):
        lo, hi = carry
        d = hi - lo
        mid = lo + (d >> 1) + (d & 1)
        ge = _count_ge(mid) >= K
        return jnp.where(ge, mid, lo), jnp.where(ge, hi, mid - 1)

    lo, hi = lax.fori_loop(0, 31, _bis, (lo, hi))
    thr = _key_f32(_splat(lo))

    def _zero(ref, val, n):
        for j in range(n):
            ref[pl.ds(j * 16, 16)] = val

    _zero(cand, _splat(0), ROWCAP // 16)

    RU = 4

    def _rowscan(g, off):
        ms = [rm[pl.ds((g * RU + u) * 16, 16)] >= thr for u in range(RU)]
        anym = ms[0] | ms[1] | ms[2] | ms[3]

        def hit(o):
            for u in range(RU):
                m = ms[u]

                def sub(o, m=m, u=u):
                    cnt = jnp.sum(m.astype(jnp.int32), axis=0)
                    plsc.store_compressed(
                        cand.at[pl.ds(jnp.minimum(o, ROWCAP - 16), 16)],
                        iota + (g * RU + u) * 16, mask=m)
                    return o + cnt

                o = lax.cond(jnp.any(m), sub, lambda o: o, o)
            return o

        return lax.cond(jnp.any(anym), hit, lambda o: o, off)

    nrows = jnp.minimum(lax.fori_loop(0, NVR // RU, _rowscan, 0), ROWCAP)

    base = b * NROW
    for j in range(ROWCAP // 16):
        gidx[j // 8, pl.ds((j % 8) * 16, 16)] = cand[pl.ds(j * 16, 16)] + base
    copies = [
        pltpu.async_copy(sc2_hbm.at[gidx.at[q]],
                         gbuf.at[pl.ds(q * 128, 128)], sem)
        for q in range(ROWCAP // 128)
    ]
    for cp in copies:
        cp.wait()

    _zero(fval, neg_inf, ELCAP // 16)
    _zero(fpos, _splat(_I32_MAX), ELCAP // 16)

    def _elscan(s, off):
        rowid = plsc.load_gather(cand, [_splat(s)])
        pos0 = rowid * W

        def inner(w, off):
            v = gbuf[s, pl.ds(w * 16, 16)]
            m = v >= thr

            def hit(o):
                cnt = jnp.sum(m.astype(jnp.int32), axis=0)
                oc = jnp.minimum(o, ELCAP - 16)
                plsc.store_compressed(fval.at[pl.ds(oc, 16)], v, mask=m)
                plsc.store_compressed(fpos.at[pl.ds(oc, 16)],
                                      pos0 + w * 16 + iota, mask=m)
                return o + cnt

            return lax.cond(jnp.any(m), hit, lambda o: o, off)

        return lax.fori_loop(0, W // 16, inner, off)

    lax.fori_loop(0, nrows, _elscan, 0)

    for i in range(NEV):
        fkey[pl.ds(i * 16, 16)] = _f32_key(fval[pl.ds(i * 16, 16)])

    kts = [fkey[pl.ds(tv * 16, 16)] for tv in range(NEV)]
    tidx = [iota + tv * 16 for tv in range(NEV)]

    def _rank(sv, accs):
        accs = list(accs)
        for lane in range(16):
            sidx = sv * 16 + lane
            ks = plsc.load_gather(fkey, [_splat(sidx)])
            si = _splat(sidx)
            for tv in range(NEV):
                earlier = si < tidx[tv]
                beats = jnp.where(earlier, ks >= kts[tv], ks > kts[tv])
                accs[tv] = accs[tv] + beats.astype(jnp.int32)
        return tuple(accs)

    ranks = lax.fori_loop(0, NEV, _rank,
                          tuple(_splat(0) for _ in range(NEV)))

    for tv in range(NEV):
        win = ranks[tv] < K
        plsc.store_scatter(sval, [ranks[tv]], fval[pl.ds(tv * 16, 16)],
                           mask=win)
        plsc.store_scatter(spos, [ranks[tv]], fpos[pl.ds(tv * 16, 16)],
                           mask=win)

    for j in range(OUTW // 16):
        sl = pl.ds(j * 16, 16)
        if j * 16 >= K:
            sval[sl] = jnp.zeros((16,), jnp.float32)
            spos[sl] = _splat(0)
        p = spos[sl]
        scls[sl] = p >> 14
        spos[sl] = p & (HW - 1)
    pltpu.sync_copy(sval, ov_hbm.at[b])
    pltpu.sync_copy(spos, oi_hbm.at[b])
    pltpu.sync_copy(scls, oc_hbm.at[b])


@jax.jit
def kernel(scores):
    rowmax = pl.pallas_call(
        _rowmax_body,
        grid=(B, C // CB),
        in_specs=[pl.BlockSpec((1, CB, H, W), lambda b, c: (b, c, 0, 0))],
        out_specs=pl.BlockSpec((1, CB, H), lambda b, c: (b, c, 0)),
        out_shape=jax.ShapeDtypeStruct((B, C, H), jnp.float32),
    )(scores)

    select = functools.partial(
        pl.kernel,
        out_type=[
            jax.ShapeDtypeStruct((B, OUTW), jnp.float32),
            jax.ShapeDtypeStruct((B, OUTW), jnp.int32),
            jax.ShapeDtypeStruct((B, OUTW), jnp.int32),
        ],
        mesh=plsc.VectorSubcoreMesh(core_axis_name="c", subcore_axis_name="s",
                                    num_cores=NCORES, num_subcores=NSUB),
        compiler_params=pltpu.CompilerParams(needs_layout_passes=False),
        scratch_types=[
            pltpu.VMEM((NROW,), jnp.float32),
            pltpu.VMEM((NGJ * 16,), jnp.int32),
            pltpu.VMEM((ROWCAP,), jnp.int32),
            pltpu.VMEM((ROWCAP // 128, 128), jnp.int32),
            pltpu.VMEM((ROWCAP, W), jnp.float32),
            pltpu.VMEM((ELCAP,), jnp.float32),
            pltpu.VMEM((ELCAP,), jnp.int32),
            pltpu.VMEM((ELCAP,), jnp.int32),
            pltpu.VMEM((OUTW,), jnp.float32),
            pltpu.VMEM((OUTW,), jnp.int32),
            pltpu.VMEM((OUTW,), jnp.int32),
            pltpu.SemaphoreType.DMA,
        ],
    )(_select_body)

    ov, oi, oc = select(rowmax.reshape(B, NROW), scores.reshape(B * NROW, W))
    return ov[:, :K], oi[:, :K], oc[:, :K]


# speedup vs baseline: 156.1449x; 1.0009x over previous
import functools

import jax
import jax.numpy as jnp
from jax import lax
from jax.experimental import pallas as pl
from jax.experimental.pallas import tpu as pltpu
from jax.experimental.pallas import tpu_sc as plsc

B, C, H, W = 32, 80, 128, 128
K = 100
HW = H * W
NROW = C * H
NVR = NROW // 16
NGJ = 32
NGT = NVR // NGJ
ROWCAP = 256
ELCAP = 192
NEV = ELCAP // 16
OUTW = 128
CB = 80
NCORES = 2
NSUB = 16

_I32_MAX = 2**31 - 1
_MASK31 = 0x7FFFFFFF


def _rowmax_body(x_ref, o_ref):
    o_ref[...] = jnp.max(x_ref[...], axis=-1)


def _f32_key(v):
    kb = lax.bitcast_convert_type(v, jnp.int32)
    return jnp.where(kb >= 0, kb, kb ^ _MASK31)


def _key_f32(k):
    return lax.bitcast_convert_type(jnp.where(k >= 0, k, k ^ _MASK31),
                                    jnp.float32)


def _splat(x, dtype=jnp.int32):
    return jnp.full((16,), x, dtype)


def _scalar(vec, is_min=False):
    return jnp.min(vec, axis=0) if is_min else jnp.max(vec, axis=0)


def _select_body(rm_hbm, sc2_hbm, ov_hbm, oi_hbm, oc_hbm,
                 rm, gk, cand, gidx, gbuf, fval, fpos, fkey,
                 sval, spos, scls, sem):
    b = lax.axis_index("s") * NCORES + lax.axis_index("c")
    iota = lax.iota(jnp.int32, 16)
    neg_inf = _splat(-jnp.inf, jnp.float32)

    with jax.named_scope("rmcopy"):
        pltpu.sync_copy(rm_hbm.at[b], rm)

    def _gmax(t, accs):
        return tuple(
            jnp.maximum(accs[j], rm[pl.ds((j + NGJ * t) * 16, 16)])
            for j in range(NGJ))

    with jax.named_scope("gmax"):
        accs = lax.fori_loop(1, NGT, _gmax,
                             tuple(rm[pl.ds(j * 16, 16)] for j in range(NGJ)))
        for j in range(NGJ):
            gk[pl.ds(j * 16, 16)] = _f32_key(accs[j])

    def _count_ge(t):
        ts = _splat(t)
        acc = (gk[pl.ds(0, 16)] >= ts).astype(jnp.int32)
        for i in range(1, NGJ):
            acc = acc + (gk[pl.ds(i * 16, 16)] >= ts).astype(jnp.int32)
        return jnp.sum(acc, axis=0)

    ge0 = _count_ge(jnp.int32(0)) >= K
    lo = jnp.where(ge0, jnp.int32(0), jnp.int32(-2**31))
    hi = jnp.where(ge0, jnp.int32(_I32_MAX), jnp.int32(-1))

    def _bis(_, carry):
        lo, hi = carry
        d = hi - lo
        mid = lo + (d >> 1) + (d & 1)
        ge = _count_ge(mid) >= K
        return jnp.where(ge, mid, lo), jnp.where(ge, hi, mid - 1)

    with jax.named_scope("bisect"):
        lo, hi = lax.fori_loop(0, 31, _bis, (lo, hi))
    thr = _key_f32(_splat(lo))

    def _zero(ref, val, n):
        for j in range(n):
            ref[pl.ds(j * 16, 16)] = val

    _zero(cand, _splat(0), ROWCAP // 16)

    RU = 4

    def _rowscan(g, off):
        ms = [rm[pl.ds((g * RU + u) * 16, 16)] >= thr for u in range(RU)]
        anym = ms[0] | ms[1] | ms[2] | ms[3]

        def hit(o):
            for u in range(RU):
                m = ms[u]

                def sub(o, m=m, u=u):
                    cnt = jnp.sum(m.astype(jnp.int32), axis=0)
                    plsc.store_compressed(
                        cand.at[pl.ds(jnp.minimum(o, ROWCAP - 16), 16)],
                        iota + (g * RU + u) * 16, mask=m)
                    return o + cnt

                o = lax.cond(jnp.any(m), sub, lambda o: o, o)
            return o

        return lax.cond(jnp.any(anym), hit, lambda o: o, off)

    with jax.named_scope("rowscan"):
        nrows = jnp.minimum(lax.fori_loop(0, NVR // RU, _rowscan, 0), ROWCAP)

    base = b * NROW
    for j in range(ROWCAP // 16):
        gidx[j // 8, pl.ds((j % 8) * 16, 16)] = cand[pl.ds(j * 16, 16)] + base
    copies = [
        pltpu.async_copy(sc2_hbm.at[gidx.at[q]],
                         gbuf.at[pl.ds(q * 128, 128)], sem)
        for q in range(ROWCAP // 128)
    ]
    with jax.named_scope("gather"):
        for cp in copies:
            cp.wait()

    _zero(fval, neg_inf, ELCAP // 16)
    _zero(fpos, _splat(_I32_MAX), ELCAP // 16)

    def _elscan(s, off):
        rowid = plsc.load_gather(cand, [_splat(s)])
        pos0 = rowid * W

        def inner(w, off):
            v = gbuf[s, pl.ds(w * 16, 16)]
            m = v >= thr

            def hit(o):
                cnt = jnp.sum(m.astype(jnp.int32), axis=0)
                oc = jnp.minimum(o, ELCAP - 16)
                plsc.store_compressed(fval.at[pl.ds(oc, 16)], v, mask=m)
                plsc.store_compressed(fpos.at[pl.ds(oc, 16)],
                                      pos0 + w * 16 + iota, mask=m)
                return o + cnt

            return lax.cond(jnp.any(m), hit, lambda o: o, off)

        return lax.fori_loop(0, W // 16, inner, off)

    with jax.named_scope("elscan"):
        lax.fori_loop(0, nrows, _elscan, 0)

    for i in range(NEV):
        fkey[pl.ds(i * 16, 16)] = _f32_key(fval[pl.ds(i * 16, 16)])

    kts = [fkey[pl.ds(tv * 16, 16)] for tv in range(NEV)]
    tidx = [iota + tv * 16 for tv in range(NEV)]

    def _rank(sv, accs):
        accs = list(accs)
        for lane in range(16):
            sidx = sv * 16 + lane
            ks = plsc.load_gather(fkey, [_splat(sidx)])
            si = _splat(sidx)
            for tv in range(NEV):
                earlier = si < tidx[tv]
                beats = jnp.where(earlier, ks >= kts[tv], ks > kts[tv])
                accs[tv] = accs[tv] + beats.astype(jnp.int32)
        return tuple(accs)

    with jax.named_scope("rank"):
        ranks = lax.fori_loop(0, NEV, _rank,
                              tuple(_splat(0) for _ in range(NEV)))

    for tv in range(NEV):
        win = ranks[tv] < K
        plsc.store_scatter(sval, [ranks[tv]], fval[pl.ds(tv * 16, 16)],
                           mask=win)
        plsc.store_scatter(spos, [ranks[tv]], fpos[pl.ds(tv * 16, 16)],
                           mask=win)

    for j in range(OUTW // 16):
        sl = pl.ds(j * 16, 16)
        if j * 16 >= K:
            sval[sl] = jnp.zeros((16,), jnp.float32)
            spos[sl] = _splat(0)
        p = spos[sl]
        scls[sl] = p >> 14
        spos[sl] = p & (HW - 1)
    pltpu.sync_copy(sval, ov_hbm.at[b])
    pltpu.sync_copy(spos, oi_hbm.at[b])
    pltpu.sync_copy(scls, oc_hbm.at[b])


@jax.jit
def kernel(scores):
    rowmax = pl.pallas_call(
        _rowmax_body,
        grid=(B, C // CB),
        in_specs=[pl.BlockSpec((1, CB, H, W), lambda b, c: (b, c, 0, 0))],
        out_specs=pl.BlockSpec((1, CB, H), lambda b, c: (b, c, 0)),
        out_shape=jax.ShapeDtypeStruct((B, C, H), jnp.float32),
    )(scores)

    select = functools.partial(
        pl.kernel,
        out_type=[
            jax.ShapeDtypeStruct((B, OUTW), jnp.float32),
            jax.ShapeDtypeStruct((B, OUTW), jnp.int32),
            jax.ShapeDtypeStruct((B, OUTW), jnp.int32),
        ],
        mesh=plsc.VectorSubcoreMesh(core_axis_name="c", subcore_axis_name="s",
                                    num_cores=NCORES, num_subcores=NSUB),
        compiler_params=pltpu.CompilerParams(needs_layout_passes=False),
        scratch_types=[
            pltpu.VMEM((NROW,), jnp.float32),
            pltpu.VMEM((NGJ * 16,), jnp.int32),
            pltpu.VMEM((ROWCAP,), jnp.int32),
            pltpu.VMEM((ROWCAP // 128, 128), jnp.int32),
            pltpu.VMEM((ROWCAP, W), jnp.float32),
            pltpu.VMEM((ELCAP,), jnp.float32),
            pltpu.VMEM((ELCAP,), jnp.int32),
            pltpu.VMEM((ELCAP,), jnp.int32),
            pltpu.VMEM((OUTW,), jnp.float32),
            pltpu.VMEM((OUTW,), jnp.int32),
            pltpu.VMEM((OUTW,), jnp.int32),
            pltpu.SemaphoreType.DMA,
        ],
    )(_select_body)

    ov, oi, oc = select(rowmax.reshape(B, NROW), scores.reshape(B * NROW, W))
    return ov[:, :K], oi[:, :K], oc[:, :K]


# speedup vs baseline: 186.7792x; 1.1962x over previous
import functools

import jax
import jax.numpy as jnp
from jax import lax
from jax.experimental import pallas as pl
from jax.experimental.pallas import tpu as pltpu
from jax.experimental.pallas import tpu_sc as plsc

B, C, H, W = 32, 80, 128, 128
K = 100
HW = H * W
NROW = C * H
NVR = NROW // 16
NGJ = 32
NGT = NVR // NGJ
ROWCAP = 256
ELCAP = 192
NEV = ELCAP // 16
OUTW = 128
CB = 80
NCORES = 2
NSUB = 16

_I32_MAX = 2**31 - 1
_MASK31 = 0x7FFFFFFF


def _rowmax_body(x_ref, o_ref):
    o_ref[...] = jnp.max(x_ref[...], axis=-1)


def _f32_key(v):
    kb = lax.bitcast_convert_type(v, jnp.int32)
    return jnp.where(kb >= 0, kb, kb ^ _MASK31)


def _key_f32(k):
    return lax.bitcast_convert_type(jnp.where(k >= 0, k, k ^ _MASK31),
                                    jnp.float32)


def _splat(x, dtype=jnp.int32):
    return jnp.full((16,), x, dtype)


def _scalar(vec, is_min=False):
    return jnp.min(vec, axis=0) if is_min else jnp.max(vec, axis=0)


def _select_body(rm_hbm, sc2_hbm, ov_hbm, oi_hbm, oc_hbm,
                 rm, gk, cand, gidx, gbuf, fval, fpos, fkey,
                 sval, spos, scls, sem):
    b = lax.axis_index("s") * NCORES + lax.axis_index("c")
    iota = lax.iota(jnp.int32, 16)
    neg_inf = _splat(-jnp.inf, jnp.float32)

    with jax.named_scope("rmcopy"):
        pltpu.sync_copy(rm_hbm.at[b], rm)

    def _gmax(t, accs):
        return tuple(
            jnp.maximum(accs[j], rm[pl.ds((j + NGJ * t) * 16, 16)])
            for j in range(NGJ))

    with jax.named_scope("gmax"):
        accs = lax.fori_loop(1, NGT, _gmax,
                             tuple(rm[pl.ds(j * 16, 16)] for j in range(NGJ)))
        for j in range(NGJ):
            gk[pl.ds(j * 16, 16)] = _f32_key(accs[j])

    def _count_ge(t):
        ts = _splat(t)
        acc = (gk[pl.ds(0, 16)] >= ts).astype(jnp.int32)
        for i in range(1, NGJ):
            acc = acc + (gk[pl.ds(i * 16, 16)] >= ts).astype(jnp.int32)
        return jnp.sum(acc, axis=0)

    ge0 = _count_ge(jnp.int32(0)) >= K
    lo = jnp.where(ge0, jnp.int32(0), jnp.int32(-2**31))
    hi = jnp.where(ge0, jnp.int32(_I32_MAX), jnp.int32(-1))

    def _bis(_, carry):
        lo, hi = carry
        d = hi - lo
        mid = lo + (d >> 1) + (d & 1)
        ge = _count_ge(mid) >= K
        return jnp.where(ge, mid, lo), jnp.where(ge, hi, mid - 1)

    with jax.named_scope("bisect"):
        lo, hi = lax.fori_loop(0, 31, _bis, (lo, hi))
    thr = _key_f32(_splat(lo))

    def _zero(ref, val, n):
        for j in range(n):
            ref[pl.ds(j * 16, 16)] = val

    _zero(cand, _splat(0), ROWCAP // 16)

    RU = 4

    def _rowscan(g, off):
        for u in range(RU):
            i = g * RU + u
            m = rm[pl.ds(i * 16, 16)] >= thr
            pos = plsc.cumsum(m.astype(jnp.int32)) + off
            idx = jnp.minimum(pos - 1, ROWCAP - 1)
            plsc.store_scatter(cand, [idx], iota + i * 16, mask=m)
            off = off + plsc.all_reduce_population_count(m)
        return off

    with jax.named_scope("rowscan"):
        offv = lax.fori_loop(0, NVR // RU, _rowscan, _splat(0))
        nrows = jnp.minimum(jnp.max(offv, axis=0), ROWCAP)

    base = b * NROW
    for j in range(ROWCAP // 16):
        gidx[j // 8, pl.ds((j % 8) * 16, 16)] = cand[pl.ds(j * 16, 16)] + base
    with jax.named_scope("gather"):
        pltpu.async_copy(sc2_hbm.at[gidx.at[0]],
                         gbuf.at[pl.ds(0, 128)], sem).wait()

        def _gather2(z):
            pltpu.async_copy(sc2_hbm.at[gidx.at[1]],
                             gbuf.at[pl.ds(128, 128)], sem).wait()
            return z

        lax.cond(nrows > 128, _gather2, lambda z: z, 0)

    _zero(fval, neg_inf, ELCAP // 16)
    _zero(fpos, _splat(_I32_MAX), ELCAP // 16)

    def _elscan(s, off):
        rowid = plsc.load_gather(cand, [_splat(s)])
        pos0 = rowid * W
        for w in range(W // 16):
            v = gbuf[s, pl.ds(w * 16, 16)]
            m = v >= thr
            pos = plsc.cumsum(m.astype(jnp.int32)) + off
            idx = jnp.minimum(pos - 1, ELCAP - 1)
            plsc.store_scatter(fval, [idx], v, mask=m)
            plsc.store_scatter(fpos, [idx], pos0 + w * 16 + iota, mask=m)
            off = off + plsc.all_reduce_population_count(m)
        return off

    with jax.named_scope("elscan"):
        lax.fori_loop(0, nrows, _elscan, _splat(0))

    for i in range(NEV):
        fkey[pl.ds(i * 16, 16)] = _f32_key(fval[pl.ds(i * 16, 16)])

    kts = [fkey[pl.ds(tv * 16, 16)] for tv in range(NEV)]
    tidx = [iota + tv * 16 for tv in range(NEV)]

    def _rank(sv, accs):
        accs = list(accs)
        for lane in range(16):
            sidx = sv * 16 + lane
            ks = plsc.load_gather(fkey, [_splat(sidx)])
            si = _splat(sidx)
            for tv in range(NEV):
                earlier = si < tidx[tv]
                beats = jnp.where(earlier, ks >= kts[tv], ks > kts[tv])
                accs[tv] = accs[tv] + beats.astype(jnp.int32)
        return tuple(accs)

    with jax.named_scope("rank"):
        ranks = lax.fori_loop(0, NEV, _rank,
                              tuple(_splat(0) for _ in range(NEV)))

    for tv in range(NEV):
        win = ranks[tv] < K
        plsc.store_scatter(sval, [ranks[tv]], fval[pl.ds(tv * 16, 16)],
                           mask=win)
        plsc.store_scatter(spos, [ranks[tv]], fpos[pl.ds(tv * 16, 16)],
                           mask=win)

    for j in range(OUTW // 16):
        sl = pl.ds(j * 16, 16)
        if j * 16 >= K:
            sval[sl] = jnp.zeros((16,), jnp.float32)
            spos[sl] = _splat(0)
        p = spos[sl]
        scls[sl] = p >> 14
        spos[sl] = p & (HW - 1)
    pltpu.sync_copy(sval, ov_hbm.at[b])
    pltpu.sync_copy(spos, oi_hbm.at[b])
    pltpu.sync_copy(scls, oc_hbm.at[b])


@jax.jit
def kernel(scores):
    rowmax = pl.pallas_call(
        _rowmax_body,
        grid=(B, C // CB),
        in_specs=[pl.BlockSpec((1, CB, H, W), lambda b, c: (b, c, 0, 0))],
        out_specs=pl.BlockSpec((1, CB, H), lambda b, c: (b, c, 0)),
        out_shape=jax.ShapeDtypeStruct((B, C, H), jnp.float32),
    )(scores)

    select = functools.partial(
        pl.kernel,
        out_type=[
            jax.ShapeDtypeStruct((B, OUTW), jnp.float32),
            jax.ShapeDtypeStruct((B, OUTW), jnp.int32),
            jax.ShapeDtypeStruct((B, OUTW), jnp.int32),
        ],
        mesh=plsc.VectorSubcoreMesh(core_axis_name="c", subcore_axis_name="s",
                                    num_cores=NCORES, num_subcores=NSUB),
        compiler_params=pltpu.CompilerParams(needs_layout_passes=False),
        scratch_types=[
            pltpu.VMEM((NROW,), jnp.float32),
            pltpu.VMEM((NGJ * 16,), jnp.int32),
            pltpu.VMEM((ROWCAP,), jnp.int32),
            pltpu.VMEM((ROWCAP // 128, 128), jnp.int32),
            pltpu.VMEM((ROWCAP, W), jnp.float32),
            pltpu.VMEM((ELCAP,), jnp.float32),
            pltpu.VMEM((ELCAP,), jnp.int32),
            pltpu.VMEM((ELCAP,), jnp.int32),
            pltpu.VMEM((OUTW,), jnp.float32),
            pltpu.VMEM((OUTW,), jnp.int32),
            pltpu.VMEM((OUTW,), jnp.int32),
            pltpu.SemaphoreType.DMA,
        ],
    )(_select_body)

    ov, oi, oc = select(rowmax.reshape(B, NROW), scores.reshape(B * NROW, W))
    return ov[:, :K], oi[:, :K], oc[:, :K]


# speedup vs baseline: 191.6227x; 1.0259x over previous
import functools

import jax
import jax.numpy as jnp
from jax import lax
from jax.experimental import pallas as pl
from jax.experimental.pallas import tpu as pltpu
from jax.experimental.pallas import tpu_sc as plsc

B, C, H, W = 32, 80, 128, 128
K = 100
HW = H * W
NROW = C * H
NVR = NROW // 16
NGJ = 32
NGT = NVR // NGJ
ROWCAP = 256
ELCAP = 192
NEV = ELCAP // 16
OUTW = 128
CB = 80
NCORES = 2
NSUB = 16

_I32_MAX = 2**31 - 1
_MASK31 = 0x7FFFFFFF


def _rowmax_body(x_ref, o_ref):
    o_ref[...] = jnp.max(x_ref[...], axis=-1)


def _f32_key(v):
    kb = lax.bitcast_convert_type(v, jnp.int32)
    return jnp.where(kb >= 0, kb, kb ^ _MASK31)


def _key_f32(k):
    return lax.bitcast_convert_type(jnp.where(k >= 0, k, k ^ _MASK31),
                                    jnp.float32)


def _splat(x, dtype=jnp.int32):
    return jnp.full((16,), x, dtype)


def _scalar(vec, is_min=False):
    return jnp.min(vec, axis=0) if is_min else jnp.max(vec, axis=0)


def _select_body(rm_hbm, sc2_hbm, ov_hbm, oi_hbm, oc_hbm,
                 rm, gk, cand, gcand, gidx, gbuf, fval, fpos, fkey,
                 sval, spos, scls, sem):
    b = lax.axis_index("s") * NCORES + lax.axis_index("c")
    iota = lax.iota(jnp.int32, 16)
    neg_inf = _splat(-jnp.inf, jnp.float32)

    with jax.named_scope("rmcopy"):
        pltpu.sync_copy(rm_hbm.at[b], rm)

    def _gmax(t, accs):
        return tuple(
            jnp.maximum(accs[j], rm[pl.ds((j + NGJ * t) * 16, 16)])
            for j in range(NGJ))

    with jax.named_scope("gmax"):
        accs = lax.fori_loop(1, NGT, _gmax,
                             tuple(rm[pl.ds(j * 16, 16)] for j in range(NGJ)))
        for j in range(NGJ):
            gk[pl.ds(j * 16, 16)] = _f32_key(accs[j])

    def _count_ge(t):
        ts = _splat(t)
        acc = (gk[pl.ds(0, 16)] >= ts).astype(jnp.int32)
        for i in range(1, NGJ):
            acc = acc + (gk[pl.ds(i * 16, 16)] >= ts).astype(jnp.int32)
        return jnp.sum(acc, axis=0)

    ge0 = _count_ge(jnp.int32(0)) >= K
    lo = jnp.where(ge0, jnp.int32(0), jnp.int32(-2**31))
    hi = jnp.where(ge0, jnp.int32(_I32_MAX), jnp.int32(-1))

    def _bis(_, carry):
        lo, hi = carry
        d = hi - lo
        mid = lo + (d >> 1) + (d & 1)
        ge = _count_ge(mid) >= K
        return jnp.where(ge, mid, lo), jnp.where(ge, hi, mid - 1)

    with jax.named_scope("bisect"):
        lo, hi = lax.fori_loop(0, 31, _bis, (lo, hi))
    thr = _key_f32(_splat(lo))

    def _zero(ref, val, n):
        for j in range(n):
            ref[pl.ds(j * 16, 16)] = val

    _zero(cand, _splat(0), ROWCAP // 16)
    _zero(gcand, _splat(0), ROWCAP // 16)

    def _gscan(i, off):
        m = gk[pl.ds(i * 16, 16)] >= _splat(lo)
        pos = plsc.cumsum(m.astype(jnp.int32)) + off
        idx = jnp.minimum(pos - 1, ROWCAP - 1)
        plsc.store_scatter(gcand, [idx], iota + i * 16, mask=m)
        return off + plsc.all_reduce_population_count(m)

    def _grow(gi, off):
        gid = plsc.load_gather(gcand, [_splat(gi)])
        rbase = (gid >> 4) * 16 + (gid & 15)
        idx0 = rbase + 512 * iota
        m0 = plsc.load_gather(rm, [idx0]) >= thr
        pos = plsc.cumsum(m0.astype(jnp.int32)) + off
        plsc.store_scatter(cand, [jnp.minimum(pos - 1, ROWCAP - 1)],
                           idx0, mask=m0)
        off = off + plsc.all_reduce_population_count(m0)
        idx1 = jnp.minimum(rbase + 512 * (iota + 16), NROW - 1)
        m1 = (plsc.load_gather(rm, [idx1]) >= thr) & (iota < NGT - 16)
        pos = plsc.cumsum(m1.astype(jnp.int32)) + off
        plsc.store_scatter(cand, [jnp.minimum(pos - 1, ROWCAP - 1)],
                           idx1, mask=m1)
        return off + plsc.all_reduce_population_count(m1)

    with jax.named_scope("rowscan"):
        goff = lax.fori_loop(0, NGJ, _gscan, _splat(0))
        ngrp = jnp.minimum(jnp.max(goff, axis=0), ROWCAP)
        offv = lax.fori_loop(0, ngrp, _grow, _splat(0))
        nrows = jnp.minimum(jnp.max(offv, axis=0), ROWCAP)

    base = b * NROW
    for j in range(ROWCAP // 16):
        gidx[j // 8, pl.ds((j % 8) * 16, 16)] = cand[pl.ds(j * 16, 16)] + base
    with jax.named_scope("gather"):
        pltpu.async_copy(sc2_hbm.at[gidx.at[0]],
                         gbuf.at[pl.ds(0, 128)], sem).wait()

        def _gather2(z):
            pltpu.async_copy(sc2_hbm.at[gidx.at[1]],
                             gbuf.at[pl.ds(128, 128)], sem).wait()
            return z

        lax.cond(nrows > 128, _gather2, lambda z: z, 0)

    _zero(fval, neg_inf, ELCAP // 16)
    _zero(fpos, _splat(_I32_MAX), ELCAP // 16)

    def _elscan(s, off):
        rowid = plsc.load_gather(cand, [_splat(s)])
        pos0 = rowid * W
        for w in range(W // 16):
            v = gbuf[s, pl.ds(w * 16, 16)]
            m = v >= thr
            pos = plsc.cumsum(m.astype(jnp.int32)) + off
            idx = jnp.minimum(pos - 1, ELCAP - 1)
            plsc.store_scatter(fval, [idx], v, mask=m)
            plsc.store_scatter(fpos, [idx], pos0 + w * 16 + iota, mask=m)
            off = off + plsc.all_reduce_population_count(m)
        return off

    with jax.named_scope("elscan"):
        lax.fori_loop(0, nrows, _elscan, _splat(0))

    for i in range(NEV):
        fkey[pl.ds(i * 16, 16)] = _f32_key(fval[pl.ds(i * 16, 16)])

    kts = [fkey[pl.ds(tv * 16, 16)] for tv in range(NEV)]
    pts = [fpos[pl.ds(tv * 16, 16)] for tv in range(NEV)]

    def _rank(sv, accs):
        accs = list(accs)
        for lane in range(16):
            sidx = sv * 16 + lane
            ks = plsc.load_gather(fkey, [_splat(sidx)])
            ps = plsc.load_gather(fpos, [_splat(sidx)])
            for tv in range(NEV):
                earlier = ps < pts[tv]
                beats = jnp.where(earlier, ks >= kts[tv], ks > kts[tv])
                accs[tv] = accs[tv] + beats.astype(jnp.int32)
        return tuple(accs)

    with jax.named_scope("rank"):
        ranks = lax.fori_loop(0, NEV, _rank,
                              tuple(_splat(0) for _ in range(NEV)))

    for tv in range(NEV):
        win = ranks[tv] < K
        plsc.store_scatter(sval, [ranks[tv]], fval[pl.ds(tv * 16, 16)],
                           mask=win)
        plsc.store_scatter(spos, [ranks[tv]], fpos[pl.ds(tv * 16, 16)],
                           mask=win)

    for j in range(OUTW // 16):
        sl = pl.ds(j * 16, 16)
        if j * 16 >= K:
            sval[sl] = jnp.zeros((16,), jnp.float32)
            spos[sl] = _splat(0)
        p = spos[sl]
        scls[sl] = p >> 14
        spos[sl] = p & (HW - 1)
    pltpu.sync_copy(sval, ov_hbm.at[b])
    pltpu.sync_copy(spos, oi_hbm.at[b])
    pltpu.sync_copy(scls, oc_hbm.at[b])


@jax.jit
def kernel(scores):
    rowmax = pl.pallas_call(
        _rowmax_body,
        grid=(B, C // CB),
        in_specs=[pl.BlockSpec((1, CB, H, W), lambda b, c: (b, c, 0, 0))],
        out_specs=pl.BlockSpec((1, CB, H), lambda b, c: (b, c, 0)),
        out_shape=jax.ShapeDtypeStruct((B, C, H), jnp.float32),
    )(scores)

    select = functools.partial(
        pl.kernel,
        out_type=[
            jax.ShapeDtypeStruct((B, OUTW), jnp.float32),
            jax.ShapeDtypeStruct((B, OUTW), jnp.int32),
            jax.ShapeDtypeStruct((B, OUTW), jnp.int32),
        ],
        mesh=plsc.VectorSubcoreMesh(core_axis_name="c", subcore_axis_name="s",
                                    num_cores=NCORES, num_subcores=NSUB),
        compiler_params=pltpu.CompilerParams(needs_layout_passes=False),
        scratch_types=[
            pltpu.VMEM((NROW,), jnp.float32),
            pltpu.VMEM((NGJ * 16,), jnp.int32),
            pltpu.VMEM((ROWCAP,), jnp.int32),
            pltpu.VMEM((ROWCAP,), jnp.int32),
            pltpu.VMEM((ROWCAP // 128, 128), jnp.int32),
            pltpu.VMEM((ROWCAP, W), jnp.float32),
            pltpu.VMEM((ELCAP,), jnp.float32),
            pltpu.VMEM((ELCAP,), jnp.int32),
            pltpu.VMEM((ELCAP,), jnp.int32),
            pltpu.VMEM((OUTW,), jnp.float32),
            pltpu.VMEM((OUTW,), jnp.int32),
            pltpu.VMEM((OUTW,), jnp.int32),
            pltpu.SemaphoreType.DMA,
        ],
    )(_select_body)

    ov, oi, oc = select(rowmax.reshape(B, NROW), scores.reshape(B * NROW, W))
    return ov[:, :K], oi[:, :K], oc[:, :K]


# speedup vs baseline: 203.1790x; 1.0603x over previous
import functools

import jax
import jax.numpy as jnp
from jax import lax
from jax.experimental import pallas as pl
from jax.experimental.pallas import tpu as pltpu
from jax.experimental.pallas import tpu_sc as plsc

B, C, H, W = 32, 80, 128, 128
K = 100
HW = H * W
NROW = C * H
NVR = NROW // 16
NGJ = 32
NGT = NVR // NGJ
ROWCAP = 256
ELCAP = 192
NEV = ELCAP // 16
OUTW = 128
CB = 80
NCORES = 2
NSUB = 16

_I32_MAX = 2**31 - 1
_MASK31 = 0x7FFFFFFF


def _rowmax_body(x_ref, o_ref):
    o_ref[...] = jnp.max(x_ref[...], axis=-1)


def _f32_key(v):
    kb = lax.bitcast_convert_type(v, jnp.int32)
    return jnp.where(kb >= 0, kb, kb ^ _MASK31)


def _key_f32(k):
    return lax.bitcast_convert_type(jnp.where(k >= 0, k, k ^ _MASK31),
                                    jnp.float32)


def _splat(x, dtype=jnp.int32):
    return jnp.full((16,), x, dtype)


def _scalar(vec, is_min=False):
    return jnp.min(vec, axis=0) if is_min else jnp.max(vec, axis=0)


def _select_body(rm_hbm, sc2_hbm, ov_hbm, oi_hbm, oc_hbm,
                 rm, gk, cand, gcand, gidx, gbuf, hsub, fval, fpos, fkey,
                 sval, spos, scls, sem):
    b = lax.axis_index("s") * NCORES + lax.axis_index("c")
    iota = lax.iota(jnp.int32, 16)
    neg_inf = _splat(-jnp.inf, jnp.float32)

    with jax.named_scope("rmcopy"):
        pltpu.sync_copy(rm_hbm.at[b], rm)

    def _gmax(t, accs):
        return tuple(
            jnp.maximum(accs[j], rm[pl.ds((j + NGJ * t) * 16, 16)])
            for j in range(NGJ))

    with jax.named_scope("gmax"):
        accs = lax.fori_loop(1, NGT, _gmax,
                             tuple(rm[pl.ds(j * 16, 16)] for j in range(NGJ)))
        for j in range(NGJ):
            gk[pl.ds(j * 16, 16)] = _f32_key(accs[j])

    def _count_ge(t):
        ts = _splat(t)
        acc = (gk[pl.ds(0, 16)] >= ts).astype(jnp.int32)
        for i in range(1, NGJ):
            acc = acc + (gk[pl.ds(i * 16, 16)] >= ts).astype(jnp.int32)
        return jnp.sum(acc, axis=0)

    ge0 = _count_ge(jnp.int32(0)) >= K
    lo = jnp.where(ge0, jnp.int32(0), jnp.int32(-2**31))
    hi = jnp.where(ge0, jnp.int32(_I32_MAX), jnp.int32(-1))

    def _bis(_, carry):
        lo, hi = carry
        d = hi - lo
        mid = lo + (d >> 1) + (d & 1)
        ge = _count_ge(mid) >= K
        return jnp.where(ge, mid, lo), jnp.where(ge, hi, mid - 1)

    with jax.named_scope("bisect"):
        lo, hi = lax.fori_loop(0, 31, _bis, (lo, hi))
    thr = _key_f32(_splat(lo))

    def _zero(ref, val, n):
        for j in range(n):
            ref[pl.ds(j * 16, 16)] = val

    _zero(cand, _splat(0), ROWCAP // 16)
    _zero(gcand, _splat(0), ROWCAP // 16)

    def _gscan(i, off):
        m = gk[pl.ds(i * 16, 16)] >= _splat(lo)
        pos = plsc.cumsum(m.astype(jnp.int32)) + off
        idx = jnp.minimum(pos - 1, ROWCAP - 1)
        plsc.store_scatter(gcand, [idx], iota + i * 16, mask=m)
        return off + plsc.all_reduce_population_count(m)

    def _grow(gi, off):
        gid = plsc.load_gather(gcand, [_splat(gi)])
        rbase = (gid >> 4) * 16 + (gid & 15)
        idx0 = rbase + 512 * iota
        m0 = plsc.load_gather(rm, [idx0]) >= thr
        pos = plsc.cumsum(m0.astype(jnp.int32)) + off
        plsc.store_scatter(cand, [jnp.minimum(pos - 1, ROWCAP - 1)],
                           idx0, mask=m0)
        off = off + plsc.all_reduce_population_count(m0)
        idx1 = jnp.minimum(rbase + 512 * (iota + 16), NROW - 1)
        m1 = (plsc.load_gather(rm, [idx1]) >= thr) & (iota < NGT - 16)
        pos = plsc.cumsum(m1.astype(jnp.int32)) + off
        plsc.store_scatter(cand, [jnp.minimum(pos - 1, ROWCAP - 1)],
                           idx1, mask=m1)
        return off + plsc.all_reduce_population_count(m1)

    with jax.named_scope("rowscan"):
        goff = lax.fori_loop(0, NGJ, _gscan, _splat(0))
        ngrp = jnp.minimum(jnp.max(goff, axis=0), ROWCAP)
        offv = lax.fori_loop(0, ngrp, _grow, _splat(0))
        nrows = jnp.minimum(jnp.max(offv, axis=0), ROWCAP)

    base = b * NROW
    for j in range(ROWCAP // 16):
        gidx[j // 8, pl.ds((j % 8) * 16, 16)] = cand[pl.ds(j * 16, 16)] + base
    with jax.named_scope("gather"):
        pltpu.async_copy(sc2_hbm.at[gidx.at[0]],
                         gbuf.at[pl.ds(0, 128)], sem).wait()

        def _gather2(z):
            pltpu.async_copy(sc2_hbm.at[gidx.at[1]],
                             gbuf.at[pl.ds(128, 128)], sem).wait()
            return z

        lax.cond(nrows > 128, _gather2, lambda z: z, 0)

    _zero(fval, neg_inf, ELCAP // 16)
    _zero(fpos, _splat(_I32_MAX), ELCAP // 16)

    _zero(hsub, _splat(0), ROWCAP // 16)
    sel_w = [iota == w for w in range(W // 16)]

    def _elA(s, off):
        cnts = [plsc.all_reduce_population_count(
                    gbuf[s, pl.ds(w * 16, 16)] >= thr)
                for w in range(W // 16)]
        flags = jnp.where(sel_w[0], cnts[0], 0)
        for w in range(1, W // 16):
            flags = flags + jnp.where(sel_w[w], cnts[w], 0)
        mh = (flags > 0) & (iota < W // 16)
        pos = plsc.cumsum(mh.astype(jnp.int32)) + off
        plsc.store_scatter(hsub, [jnp.minimum(pos - 1, ROWCAP - 1)],
                           _splat(s) * 8 + iota, mask=mh)
        return off + plsc.all_reduce_population_count(mh)

    def _elB(i, off):
        hid = plsc.load_gather(hsub, [_splat(i)])
        srow = hid >> 3
        lidx = (hid & 7) * 16 + iota
        rowid = plsc.load_gather(cand, [srow])
        v = plsc.load_gather(gbuf, [srow, lidx])
        m = v >= thr
        pos = plsc.cumsum(m.astype(jnp.int32)) + off
        idx = jnp.minimum(pos - 1, ELCAP - 1)
        plsc.store_scatter(fval, [idx], v, mask=m)
        plsc.store_scatter(fpos, [idx], rowid * W + lidx, mask=m)
        return off + plsc.all_reduce_population_count(m)

    with jax.named_scope("elscan"):
        hoff = lax.fori_loop(0, nrows, _elA, _splat(0))
        nh = jnp.minimum(jnp.max(hoff, axis=0), ROWCAP)
        lax.fori_loop(0, nh, _elB, _splat(0))

    for i in range(NEV):
        fkey[pl.ds(i * 16, 16)] = _f32_key(fval[pl.ds(i * 16, 16)])

    kts = [fkey[pl.ds(tv * 16, 16)] for tv in range(NEV)]
    pts = [fpos[pl.ds(tv * 16, 16)] for tv in range(NEV)]

    def _rank(sv, accs):
        accs = list(accs)
        for lane in range(16):
            sidx = sv * 16 + lane
            ks = plsc.load_gather(fkey, [_splat(sidx)])
            ps = plsc.load_gather(fpos, [_splat(sidx)])
            for tv in range(NEV):
                earlier = ps < pts[tv]
                beats = jnp.where(earlier, ks >= kts[tv], ks > kts[tv])
                accs[tv] = accs[tv] + beats.astype(jnp.int32)
        return tuple(accs)

    with jax.named_scope("rank"):
        ranks = lax.fori_loop(0, NEV, _rank,
                              tuple(_splat(0) for _ in range(NEV)))

    for tv in range(NEV):
        win = ranks[tv] < K
        plsc.store_scatter(sval, [ranks[tv]], fval[pl.ds(tv * 16, 16)],
                           mask=win)
        plsc.store_scatter(spos, [ranks[tv]], fpos[pl.ds(tv * 16, 16)],
                           mask=win)

    for j in range(OUTW // 16):
        sl = pl.ds(j * 16, 16)
        if j * 16 >= K:
            sval[sl] = jnp.zeros((16,), jnp.float32)
            spos[sl] = _splat(0)
        p = spos[sl]
        scls[sl] = p >> 14
        spos[sl] = p & (HW - 1)
    pltpu.sync_copy(sval, ov_hbm.at[b])
    pltpu.sync_copy(spos, oi_hbm.at[b])
    pltpu.sync_copy(scls, oc_hbm.at[b])


@jax.jit
def kernel(scores):
    rowmax = pl.pallas_call(
        _rowmax_body,
        grid=(B, C // CB),
        in_specs=[pl.BlockSpec((1, CB, H, W), lambda b, c: (b, c, 0, 0))],
        out_specs=pl.BlockSpec((1, CB, H), lambda b, c: (b, c, 0)),
        out_shape=jax.ShapeDtypeStruct((B, C, H), jnp.float32),
    )(scores)

    select = functools.partial(
        pl.kernel,
        out_type=[
            jax.ShapeDtypeStruct((B, OUTW), jnp.float32),
            jax.ShapeDtypeStruct((B, OUTW), jnp.int32),
            jax.ShapeDtypeStruct((B, OUTW), jnp.int32),
        ],
        mesh=plsc.VectorSubcoreMesh(core_axis_name="c", subcore_axis_name="s",
                                    num_cores=NCORES, num_subcores=NSUB),
        compiler_params=pltpu.CompilerParams(needs_layout_passes=False),
        scratch_types=[
            pltpu.VMEM((NROW,), jnp.float32),
            pltpu.VMEM((NGJ * 16,), jnp.int32),
            pltpu.VMEM((ROWCAP,), jnp.int32),
            pltpu.VMEM((ROWCAP,), jnp.int32),
            pltpu.VMEM((ROWCAP // 128, 128), jnp.int32),
            pltpu.VMEM((ROWCAP, W), jnp.float32),
            pltpu.VMEM((ROWCAP,), jnp.int32),
            pltpu.VMEM((ELCAP,), jnp.float32),
            pltpu.VMEM((ELCAP,), jnp.int32),
            pltpu.VMEM((ELCAP,), jnp.int32),
            pltpu.VMEM((OUTW,), jnp.float32),
            pltpu.VMEM((OUTW,), jnp.int32),
            pltpu.VMEM((OUTW,), jnp.int32),
            pltpu.SemaphoreType.DMA,
        ],
    )(_select_body)

    ov, oi, oc = select(rowmax.reshape(B, NROW), scores.reshape(B * NROW, W))
    return ov[:, :K], oi[:, :K], oc[:, :K]
